# Initial kernel scaffold; baseline (speedup 1.0000x reference)
import jax
import jax.numpy as jnp
from jax.experimental import pallas as pl


def kernel(sparse_fea, dense_fea, params):
    raise NotImplementedError("write your pallas kernel here")


# speedup vs baseline: 5.6900x; 5.6900x over previous
import jax
import jax.numpy as jnp
from jax import lax
from jax.experimental import pallas as pl

_F32 = jnp.float32


def _wt(w, b):
    return w.T, b.reshape(1, -1)


def _mmg(x, w_t, b):
    bs, n, c = x.shape
    o = w_t.shape[1]

    def body(x_ref, w_ref, b_ref, o_ref):
        o_ref[0] = jax.nn.gelu(
            jnp.dot(x_ref[0], w_ref[...], preferred_element_type=_F32)
            + b_ref[...])

    return pl.pallas_call(
        body,
        grid=(bs,),
        in_specs=[
            pl.BlockSpec((1, n, c), lambda i: (i, 0, 0)),
            pl.BlockSpec((c, o), lambda i: (0, 0)),
            pl.BlockSpec((1, o), lambda i: (0, 0)),
        ],
        out_specs=pl.BlockSpec((1, n, o), lambda i: (i, 0, 0)),
        out_shape=jax.ShapeDtypeStruct((bs, n, o), _F32),
    )(x, w_t, b)


_BF16 = jnp.bfloat16


def _split3(xf):
    hi = xf.astype(_BF16)
    r1 = xf - hi.astype(_F32)
    mid = r1.astype(_BF16)
    lo = (r1 - mid.astype(_F32)).astype(_BF16)
    return hi, mid, lo


def _exact_gather(oh_bf, parts):
    hi, mid, lo = parts
    nb = jnp.dot(oh_bf, hi, preferred_element_type=_F32)
    nb = nb + jnp.dot(oh_bf, mid, preferred_element_type=_F32)
    return nb + jnp.dot(oh_bf, lo, preferred_element_type=_F32)


def _edge_gcn(x, sq_row, sq_col, w1_t, b1, w2_t, b2, k, tile):
    bs, n, c = x.shape
    p = w1_t.shape[1]
    q = w2_t.shape[1]

    def body(xt_ref, xf_ref, sqr_ref, sqc_ref, w1_ref, b1_ref, w2_ref,
             b2_ref, o_ref):
        xt = xt_ref[0]
        xf = xf_ref[0]
        parts = _split3(xf)
        g = lax.dot_general(xt, xf, (((1,), (1,)), ((), ())),
                            preferred_element_type=_F32)
        dist = (sqr_ref[0] - 2.0 * g) + sqc_ref[0]
        iota = lax.broadcasted_iota(jnp.int32, (tile, n), 1)
        acc = jnp.full((tile, q), -jnp.inf, dtype=_F32)
        for _ in range(k):
            m = jnp.min(dist, axis=1, keepdims=True)
            idxm = jnp.min(jnp.where(dist <= m, iota, n), axis=1,
                           keepdims=True)
            oh = iota == idxm
            nb = _exact_gather(oh.astype(_BF16), parts)
            feat = jnp.concatenate([nb - xt, xt], axis=1)
            h = jax.nn.gelu(
                jnp.dot(feat, w1_ref[...], preferred_element_type=_F32)
                + b1_ref[...])
            gl = jax.nn.gelu(jnp.dot(h, w2_ref[...],
                                     preferred_element_type=_F32)
                             + b2_ref[...])
            acc = jnp.maximum(acc, gl)
            dist = jnp.where(oh, jnp.inf, dist)
        o_ref[0] = acc

    return pl.pallas_call(
        body,
        grid=(bs, n // tile),
        in_specs=[
            pl.BlockSpec((1, tile, c), lambda i, j: (i, j, 0)),
            pl.BlockSpec((1, n, c), lambda i, j: (i, 0, 0)),
            pl.BlockSpec((1, tile, 1), lambda i, j: (i, j, 0)),
            pl.BlockSpec((1, 1, n), lambda i, j: (i, 0, 0)),
            pl.BlockSpec((2 * c, p), lambda i, j: (0, 0)),
            pl.BlockSpec((1, p), lambda i, j: (0, 0)),
            pl.BlockSpec((p, q), lambda i, j: (0, 0)),
            pl.BlockSpec((1, q), lambda i, j: (0, 0)),
        ],
        out_specs=pl.BlockSpec((1, tile, q), lambda i, j: (i, j, 0)),
        out_shape=jax.ShapeDtypeStruct((bs, n, q), _F32),
    )(x, x, sq_row, sq_col, w1_t, b1, w2_t, b2)


def _stage1(d5, s, wt_cat, bt, w_d2s_t, b_d2s, n_stk, n_pnt):
    bs, n, c5 = d5.shape
    cd = wt_cat.shape[1]
    cs = s.shape[2]
    u = w_d2s_t.shape[1]

    def body(d5_ref, s_ref, wt_ref, bt_ref, wd2s_ref, bd_ref, us_ref):
        t = jax.nn.gelu(
            jnp.dot(d5_ref[0], wt_ref[...], preferred_element_type=_F32)
            + bt_ref[...])
        spdn = jnp.max(t.reshape(n_stk, n_pnt, cd), axis=1)
        cat = jnp.concatenate([s_ref[0], spdn], axis=1)
        us_ref[0] = jax.nn.gelu(
            jnp.dot(cat, wd2s_ref[...], preferred_element_type=_F32)
            + bd_ref[...])

    return pl.pallas_call(
        body,
        grid=(bs,),
        in_specs=[
            pl.BlockSpec((1, n, c5), lambda i: (i, 0, 0)),
            pl.BlockSpec((1, n_stk, cs), lambda i: (i, 0, 0)),
            pl.BlockSpec((c5, cd), lambda i: (0, 0)),
            pl.BlockSpec((1, cd), lambda i: (0, 0)),
            pl.BlockSpec((cs + cd, u), lambda i: (0, 0)),
            pl.BlockSpec((1, u), lambda i: (0, 0)),
        ],
        out_specs=pl.BlockSpec((1, n_stk, u), lambda i: (i, 0, 0)),
        out_shape=jax.ShapeDtypeStruct((bs, n_stk, u), _F32),
    )(d5, s, wt_cat, bt, w_d2s_t, b_d2s)


def _sparse_gcn(x, sq_row, sq_col, wts, k):
    bs, n, c = x.shape
    (w11, b11, w12, b12, w21, b21, w22, b22, w31, b31, w32, b32) = wts
    q_out = w32.shape[1]

    def gcn_block(xv, sq_t, sq_f, w1_ref, b1_ref, w2_ref, b2_ref):
        parts = _split3(xv)
        if sq_t is None:
            sq_t = jnp.sum(xv * xv, axis=1, keepdims=True)
            ones_c = jnp.ones((1, xv.shape[1]), dtype=_F32)
            y = xv * xv
            yh = y.astype(_BF16)
            r1 = y - yh.astype(_F32)
            ym = r1.astype(_BF16)
            yl = (r1 - ym.astype(_F32)).astype(_BF16)
            ones_b = ones_c.astype(_BF16)
            sq_f = (lax.dot_general(ones_b, yh, (((1,), (1,)), ((), ())),
                                    preferred_element_type=_F32)
                    + lax.dot_general(ones_b, ym, (((1,), (1,)), ((), ())),
                                      preferred_element_type=_F32)
                    + lax.dot_general(ones_b, yl, (((1,), (1,)), ((), ())),
                                      preferred_element_type=_F32))
        g = lax.dot_general(xv, xv, (((1,), (1,)), ((), ())),
                            preferred_element_type=_F32)
        dist = (sq_t - 2.0 * g) + sq_f
        iota = lax.broadcasted_iota(jnp.int32, dist.shape, 1)
        q = w2_ref.shape[1]
        acc = jnp.full((xv.shape[0], q), -jnp.inf, dtype=_F32)
        for _ in range(k):
            m = jnp.min(dist, axis=1, keepdims=True)
            idxm = jnp.min(jnp.where(dist <= m, iota, dist.shape[1]), axis=1,
                           keepdims=True)
            oh = iota == idxm
            nb = _exact_gather(oh.astype(_BF16), parts)
            feat = jnp.concatenate([nb - xv, xv], axis=1)
            h = jax.nn.gelu(
                jnp.dot(feat, w1_ref[...], preferred_element_type=_F32)
                + b1_ref[...])
            gl = jax.nn.gelu(jnp.dot(h, w2_ref[...],
                                     preferred_element_type=_F32)
                             + b2_ref[...])
            acc = jnp.maximum(acc, gl)
            dist = jnp.where(oh, jnp.inf, dist)
        return acc

    def body(x_ref, sqr_ref, sqc_ref, w11_r, b11_r, w12_r, b12_r, w21_r,
             b21_r, w22_r, b22_r, w31_r, b31_r, w32_r, b32_r, o_ref):
        xv = x_ref[0]
        x1 = gcn_block(xv, sqr_ref[0], sqc_ref[0], w11_r, b11_r, w12_r, b12_r)
        x2 = gcn_block(x1, None, None, w21_r, b21_r, w22_r, b22_r)
        xc = jnp.concatenate([x1, x2], axis=1)
        y = jax.nn.gelu(
            jnp.dot(xc, w31_r[...], preferred_element_type=_F32)
            + b31_r[...])
        o_ref[0] = jax.nn.gelu(
            jnp.dot(y, w32_r[...], preferred_element_type=_F32) + b32_r[...])

    w_specs = [pl.BlockSpec(w.shape, lambda i: (0, 0)) for w in wts]
    return pl.pallas_call(
        body,
        grid=(bs,),
        in_specs=[pl.BlockSpec((1, n, c), lambda i: (i, 0, 0)),
                  pl.BlockSpec((1, n, 1), lambda i: (i, 0, 0)),
                  pl.BlockSpec((1, 1, n), lambda i: (i, 0, 0))] + w_specs,
        out_specs=pl.BlockSpec((1, n, q_out), lambda i: (i, 0, 0)),
        out_shape=jax.ShapeDtypeStruct((bs, n, q_out), _F32),
    )(x, sq_row, sq_col, *wts)


def kernel(sparse_fea, dense_fea, params):
    bs, c_dn, n_stk, n_pnt = dense_fea.shape
    c_sp = sparse_fea.shape[1]
    n = n_stk * n_pnt

    s_pm = jnp.transpose(sparse_fea, (0, 2, 1))
    d_pm = jnp.transpose(dense_fea.reshape(bs, c_dn, n), (0, 2, 1))

    d_pad = jnp.pad(d_pm, ((0, 0), (2, 2), (0, 0)), mode='edge')
    d5 = jnp.concatenate([d_pad[:, j:j + n, :] for j in range(5)], axis=2)
    wt_cat = jnp.transpose(params['d2s_tconv_w'], (2, 1, 0)).reshape(-1, c_dn)
    bt = params['d2s_tconv_b'].reshape(1, -1)
    w_d2s, b_d2s = params['d2s_mlp'][0]
    w_s2d, b_s2d = params['s2d_mlp'][0]
    u_s = _stage1(d5, s_pm, wt_cat, bt, w_d2s.T, b_d2s.reshape(1, -1),
                  n_stk, n_pnt)
    rep = jnp.broadcast_to(s_pm[:, :, None, :],
                           (bs, n_stk, n_pnt, c_sp)).reshape(bs, n, c_sp)
    cat_d = jnp.concatenate([d_pm, rep], axis=2)
    u_d = _mmg(cat_d, w_s2d.T, b_s2d.reshape(1, -1))

    def edge_prep(layers):
        (w1, b1), (w2, b2) = layers
        return _wt(w1, b1) + _wt(w2, b2)

    sp = params['sp']
    sp_wts = edge_prep(sp['c1']) + edge_prep(sp['c2']) + edge_prep(sp['c3'])
    sq_us = jnp.sum(u_s * u_s, axis=-1)
    us = _sparse_gcn(u_s, sq_us[:, :, None], sq_us[:, None, :], sp_wts, 2)
    us_out = jnp.transpose(us, (0, 2, 1))

    dn = params['dn']
    sq_ud = jnp.sum(u_d * u_d, axis=-1)
    x1 = _edge_gcn(u_d, sq_ud[:, :, None], sq_ud[:, None, :],
                   *(_wt(*dn['c1'][0]) + _wt(*dn['c1'][1])), 10, 256)
    sq_x1 = jnp.sum(x1 * x1, axis=-1)
    x2 = _edge_gcn(x1, sq_x1[:, :, None], sq_x1[:, None, :],
                   *(_wt(*dn['c2'][0]) + _wt(*dn['c2'][1])), 10, 256)

    (w31d, b31d), (w32d, b32d) = dn['c3']
    xcat = jnp.concatenate([x1, x2], axis=2)
    y = _mmg(xcat, w31d.T, b31d.reshape(1, -1))
    yd = _mmg(y, w32d.T, b32d.reshape(1, -1))

    y_pad = jnp.pad(yd, ((0, 0), (2, 2), (0, 0)), mode='edge')
    cat6 = jnp.concatenate([y_pad[:, j:j + n - 1:2, :] for j in range(6)],
                           axis=2)
    w_ds = jnp.transpose(params['ds_conv_w'], (2, 1, 0)).reshape(-1, 128)
    ud_pm = _mmg(cat6, w_ds, params['ds_conv_b'].reshape(1, -1))
    ud_out = jnp.transpose(ud_pm, (0, 2, 1)).reshape(bs, -1, n_stk,
                                                     n_pnt // 2)
    return us_out, ud_out


# speedup vs baseline: 7.1884x; 1.2633x over previous
import jax
import jax.numpy as jnp
from jax import lax
from jax.experimental import pallas as pl

_F32 = jnp.float32


def _wt(w, b):
    return w.T, b.reshape(1, -1)


def _mmg(x, w_t, b):
    bs, n, c = x.shape
    o = w_t.shape[1]

    def body(x_ref, w_ref, b_ref, o_ref):
        o_ref[0] = jax.nn.gelu(
            jnp.dot(x_ref[0], w_ref[...], preferred_element_type=_F32)
            + b_ref[...])

    return pl.pallas_call(
        body,
        grid=(bs,),
        in_specs=[
            pl.BlockSpec((1, n, c), lambda i: (i, 0, 0)),
            pl.BlockSpec((c, o), lambda i: (0, 0)),
            pl.BlockSpec((1, o), lambda i: (0, 0)),
        ],
        out_specs=pl.BlockSpec((1, n, o), lambda i: (i, 0, 0)),
        out_shape=jax.ShapeDtypeStruct((bs, n, o), _F32),
    )(x, w_t, b)


_BF16 = jnp.bfloat16


def _split3(xf):
    hi = xf.astype(_BF16)
    r1 = xf - hi.astype(_F32)
    mid = r1.astype(_BF16)
    lo = (r1 - mid.astype(_F32)).astype(_BF16)
    return hi, mid, lo


def _exact_gather(oh_bf, parts):
    hi, mid, lo = parts
    nb = jnp.dot(oh_bf, hi, preferred_element_type=_F32)
    nb = nb + jnp.dot(oh_bf, mid, preferred_element_type=_F32)
    return nb + jnp.dot(oh_bf, lo, preferred_element_type=_F32)


def _edge_gcn(x, sq_row, sq_col, w1_t, b1, w2_t, b2, k, tile):
    bs, n, c = x.shape
    p = w1_t.shape[1]
    q = w2_t.shape[1]

    def body(xt_ref, xf_ref, sqr_ref, sqc_ref, w1_ref, b1_ref, w2_ref,
             b2_ref, o_ref):
        xt = xt_ref[0]
        xf = xf_ref[0]
        parts = _split3(xf)
        g = lax.dot_general(xt, xf, (((1,), (1,)), ((), ())),
                            preferred_element_type=_F32)
        dist = (sqr_ref[0] - 2.0 * g) + sqc_ref[0]
        iota = lax.broadcasted_iota(jnp.int32, (tile, n), 1)
        acc = jnp.full((tile, q), -jnp.inf, dtype=_F32)
        for _ in range(k):
            m = jnp.min(dist, axis=1, keepdims=True)
            idxm = jnp.min(jnp.where(dist <= m, iota, n), axis=1,
                           keepdims=True)
            oh = iota == idxm
            nb = _exact_gather(oh.astype(_BF16), parts)
            feat = jnp.concatenate([nb - xt, xt], axis=1)
            h = jax.nn.gelu(
                jnp.dot(feat, w1_ref[...], preferred_element_type=_F32)
                + b1_ref[...])
            gl = jax.nn.gelu(jnp.dot(h, w2_ref[...],
                                     preferred_element_type=_F32)
                             + b2_ref[...])
            acc = jnp.maximum(acc, gl)
            dist = jnp.where(oh, jnp.inf, dist)
        o_ref[0] = acc

    return pl.pallas_call(
        body,
        grid=(bs, n // tile),
        in_specs=[
            pl.BlockSpec((1, tile, c), lambda i, j: (i, j, 0)),
            pl.BlockSpec((1, n, c), lambda i, j: (i, 0, 0)),
            pl.BlockSpec((1, tile, 1), lambda i, j: (i, j, 0)),
            pl.BlockSpec((1, 1, n), lambda i, j: (i, 0, 0)),
            pl.BlockSpec((2 * c, p), lambda i, j: (0, 0)),
            pl.BlockSpec((1, p), lambda i, j: (0, 0)),
            pl.BlockSpec((p, q), lambda i, j: (0, 0)),
            pl.BlockSpec((1, q), lambda i, j: (0, 0)),
        ],
        out_specs=pl.BlockSpec((1, tile, q), lambda i, j: (i, j, 0)),
        out_shape=jax.ShapeDtypeStruct((bs, n, q), _F32),
    )(x, x, sq_row, sq_col, w1_t, b1, w2_t, b2)


def _edge_gcn_fast(x, sq_row, sq_col, wa_t, wcm_t, b1, w2_t, b2, k, tile):
    bs, n, c = x.shape
    p = wa_t.shape[1]
    q = w2_t.shape[1]

    def body(xt_ref, xf_ref, sqr_ref, sqc_ref, wa_ref, wc_ref, b1_ref,
             w2_ref, b2_ref, o_ref):
        xt = xt_ref[0]
        xf = xf_ref[0]
        g = lax.dot_general(xt, xf, (((1,), (1,)), ((), ())),
                            preferred_element_type=_F32)
        dist = (sqr_ref[0] - 2.0 * g) + sqc_ref[0]
        a_full = jnp.dot(xf, wa_ref[...],
                         preferred_element_type=_F32).astype(_BF16)
        b_tile = (jnp.dot(xt, wc_ref[...], preferred_element_type=_F32)
                  + b1_ref[...])
        iota = lax.broadcasted_iota(jnp.int32, (tile, n), 1)
        acc = jnp.full((tile, q), -jnp.inf, dtype=_F32)
        for _ in range(k):
            m = jnp.min(dist, axis=1, keepdims=True)
            idxm = jnp.min(jnp.where(dist <= m, iota, n), axis=1,
                           keepdims=True)
            oh = iota == idxm
            nb_a = jnp.dot(oh.astype(_BF16), a_full,
                           preferred_element_type=_F32)
            h = jax.nn.gelu(nb_a + b_tile)
            gl = jax.nn.gelu(jnp.dot(h, w2_ref[...],
                                     preferred_element_type=_F32)
                             + b2_ref[...])
            acc = jnp.maximum(acc, gl)
            dist = jnp.where(oh, jnp.inf, dist)
        o_ref[0] = acc

    return pl.pallas_call(
        body,
        grid=(bs, n // tile),
        in_specs=[
            pl.BlockSpec((1, tile, c), lambda i, j: (i, j, 0)),
            pl.BlockSpec((1, n, c), lambda i, j: (i, 0, 0)),
            pl.BlockSpec((1, tile, 1), lambda i, j: (i, j, 0)),
            pl.BlockSpec((1, 1, n), lambda i, j: (i, 0, 0)),
            pl.BlockSpec((c, p), lambda i, j: (0, 0)),
            pl.BlockSpec((c, p), lambda i, j: (0, 0)),
            pl.BlockSpec((1, p), lambda i, j: (0, 0)),
            pl.BlockSpec((p, q), lambda i, j: (0, 0)),
            pl.BlockSpec((1, q), lambda i, j: (0, 0)),
        ],
        out_specs=pl.BlockSpec((1, tile, q), lambda i, j: (i, j, 0)),
        out_shape=jax.ShapeDtypeStruct((bs, n, q), _F32),
    )(x, x, sq_row, sq_col, wa_t, wcm_t, b1, w2_t, b2)


def _stage1(d5, s, wt_cat, bt, w_d2s_t, b_d2s, n_stk, n_pnt):
    bs, n, c5 = d5.shape
    cd = wt_cat.shape[1]
    cs = s.shape[2]
    u = w_d2s_t.shape[1]

    def body(d5_ref, s_ref, wt_ref, bt_ref, wd2s_ref, bd_ref, us_ref):
        t = jax.nn.gelu(
            jnp.dot(d5_ref[0], wt_ref[...], preferred_element_type=_F32)
            + bt_ref[...])
        spdn = jnp.max(t.reshape(n_stk, n_pnt, cd), axis=1)
        cat = jnp.concatenate([s_ref[0], spdn], axis=1)
        us_ref[0] = jax.nn.gelu(
            jnp.dot(cat, wd2s_ref[...], preferred_element_type=_F32)
            + bd_ref[...])

    return pl.pallas_call(
        body,
        grid=(bs,),
        in_specs=[
            pl.BlockSpec((1, n, c5), lambda i: (i, 0, 0)),
            pl.BlockSpec((1, n_stk, cs), lambda i: (i, 0, 0)),
            pl.BlockSpec((c5, cd), lambda i: (0, 0)),
            pl.BlockSpec((1, cd), lambda i: (0, 0)),
            pl.BlockSpec((cs + cd, u), lambda i: (0, 0)),
            pl.BlockSpec((1, u), lambda i: (0, 0)),
        ],
        out_specs=pl.BlockSpec((1, n_stk, u), lambda i: (i, 0, 0)),
        out_shape=jax.ShapeDtypeStruct((bs, n_stk, u), _F32),
    )(d5, s, wt_cat, bt, w_d2s_t, b_d2s)


def _sparse_gcn(x, sq_row, sq_col, wts, k):
    bs, n, c = x.shape
    (w11, b11, w12, b12, w21, b21, w22, b22, w31, b31, w32, b32) = wts
    q_out = w32.shape[1]

    def gcn_block(xv, sq_t, sq_f, w1_ref, b1_ref, w2_ref, b2_ref):
        parts = _split3(xv)
        if sq_t is None:
            sq_t = jnp.sum(xv * xv, axis=1, keepdims=True)
            ones_c = jnp.ones((1, xv.shape[1]), dtype=_F32)
            y = xv * xv
            yh = y.astype(_BF16)
            r1 = y - yh.astype(_F32)
            ym = r1.astype(_BF16)
            yl = (r1 - ym.astype(_F32)).astype(_BF16)
            ones_b = ones_c.astype(_BF16)
            sq_f = (lax.dot_general(ones_b, yh, (((1,), (1,)), ((), ())),
                                    preferred_element_type=_F32)
                    + lax.dot_general(ones_b, ym, (((1,), (1,)), ((), ())),
                                      preferred_element_type=_F32)
                    + lax.dot_general(ones_b, yl, (((1,), (1,)), ((), ())),
                                      preferred_element_type=_F32))
        g = lax.dot_general(xv, xv, (((1,), (1,)), ((), ())),
                            preferred_element_type=_F32)
        dist = (sq_t - 2.0 * g) + sq_f
        iota = lax.broadcasted_iota(jnp.int32, dist.shape, 1)
        q = w2_ref.shape[1]
        acc = jnp.full((xv.shape[0], q), -jnp.inf, dtype=_F32)
        for _ in range(k):
            m = jnp.min(dist, axis=1, keepdims=True)
            idxm = jnp.min(jnp.where(dist <= m, iota, dist.shape[1]), axis=1,
                           keepdims=True)
            oh = iota == idxm
            nb = _exact_gather(oh.astype(_BF16), parts)
            feat = jnp.concatenate([nb - xv, xv], axis=1)
            h = jax.nn.gelu(
                jnp.dot(feat, w1_ref[...], preferred_element_type=_F32)
                + b1_ref[...])
            gl = jax.nn.gelu(jnp.dot(h, w2_ref[...],
                                     preferred_element_type=_F32)
                             + b2_ref[...])
            acc = jnp.maximum(acc, gl)
            dist = jnp.where(oh, jnp.inf, dist)
        return acc

    def body(x_ref, sqr_ref, sqc_ref, w11_r, b11_r, w12_r, b12_r, w21_r,
             b21_r, w22_r, b22_r, w31_r, b31_r, w32_r, b32_r, o_ref):
        xv = x_ref[0]
        x1 = gcn_block(xv, sqr_ref[0], sqc_ref[0], w11_r, b11_r, w12_r, b12_r)
        x2 = gcn_block(x1, None, None, w21_r, b21_r, w22_r, b22_r)
        xc = jnp.concatenate([x1, x2], axis=1)
        y = jax.nn.gelu(
            jnp.dot(xc, w31_r[...], preferred_element_type=_F32)
            + b31_r[...])
        o_ref[0] = jax.nn.gelu(
            jnp.dot(y, w32_r[...], preferred_element_type=_F32) + b32_r[...])

    w_specs = [pl.BlockSpec(w.shape, lambda i: (0, 0)) for w in wts]
    return pl.pallas_call(
        body,
        grid=(bs,),
        in_specs=[pl.BlockSpec((1, n, c), lambda i: (i, 0, 0)),
                  pl.BlockSpec((1, n, 1), lambda i: (i, 0, 0)),
                  pl.BlockSpec((1, 1, n), lambda i: (i, 0, 0))] + w_specs,
        out_specs=pl.BlockSpec((1, n, q_out), lambda i: (i, 0, 0)),
        out_shape=jax.ShapeDtypeStruct((bs, n, q_out), _F32),
    )(x, sq_row, sq_col, *wts)


def kernel(sparse_fea, dense_fea, params):
    bs, c_dn, n_stk, n_pnt = dense_fea.shape
    c_sp = sparse_fea.shape[1]
    n = n_stk * n_pnt

    s_pm = jnp.transpose(sparse_fea, (0, 2, 1))
    d_pm = jnp.transpose(dense_fea.reshape(bs, c_dn, n), (0, 2, 1))

    d_pad = jnp.pad(d_pm, ((0, 0), (2, 2), (0, 0)), mode='edge')
    d5 = jnp.concatenate([d_pad[:, j:j + n, :] for j in range(5)], axis=2)
    wt_cat = jnp.transpose(params['d2s_tconv_w'], (2, 1, 0)).reshape(-1, c_dn)
    bt = params['d2s_tconv_b'].reshape(1, -1)
    w_d2s, b_d2s = params['d2s_mlp'][0]
    w_s2d, b_s2d = params['s2d_mlp'][0]
    u_s = _stage1(d5, s_pm, wt_cat, bt, w_d2s.T, b_d2s.reshape(1, -1),
                  n_stk, n_pnt)
    rep = jnp.broadcast_to(s_pm[:, :, None, :],
                           (bs, n_stk, n_pnt, c_sp)).reshape(bs, n, c_sp)
    cat_d = jnp.concatenate([d_pm, rep], axis=2)
    u_d = _mmg(cat_d, w_s2d.T, b_s2d.reshape(1, -1))

    def edge_prep(layers):
        (w1, b1), (w2, b2) = layers
        return _wt(w1, b1) + _wt(w2, b2)

    sp = params['sp']
    sp_wts = edge_prep(sp['c1']) + edge_prep(sp['c2']) + edge_prep(sp['c3'])
    sq_us = jnp.sum(u_s * u_s, axis=-1)
    us = _sparse_gcn(u_s, sq_us[:, :, None], sq_us[:, None, :], sp_wts, 2)
    us_out = jnp.transpose(us, (0, 2, 1))

    dn = params['dn']
    sq_ud = jnp.sum(u_d * u_d, axis=-1)
    x1 = _edge_gcn(u_d, sq_ud[:, :, None], sq_ud[:, None, :],
                   *(_wt(*dn['c1'][0]) + _wt(*dn['c1'][1])), 10, 256)
    sq_x1 = jnp.sum(x1 * x1, axis=-1)
    w21, b21 = dn['c2'][0]
    c_x1 = x1.shape[2]
    x2 = _edge_gcn_fast(x1, sq_x1[:, :, None], sq_x1[:, None, :],
                        w21[:, :c_x1].T, (w21[:, c_x1:] - w21[:, :c_x1]).T,
                        b21.reshape(1, -1), *_wt(*dn['c2'][1]), 10, 256)

    (w31d, b31d), (w32d, b32d) = dn['c3']
    xcat = jnp.concatenate([x1, x2], axis=2)
    y = _mmg(xcat, w31d.T, b31d.reshape(1, -1))
    yd = _mmg(y, w32d.T, b32d.reshape(1, -1))

    y_pad = jnp.pad(yd, ((0, 0), (2, 2), (0, 0)), mode='edge')
    cat6 = jnp.concatenate([y_pad[:, j:j + n - 1:2, :] for j in range(6)],
                           axis=2)
    w_ds = jnp.transpose(params['ds_conv_w'], (2, 1, 0)).reshape(-1, 128)
    ud_pm = _mmg(cat6, w_ds, params['ds_conv_b'].reshape(1, -1))
    ud_out = jnp.transpose(ud_pm, (0, 2, 1)).reshape(bs, -1, n_stk,
                                                     n_pnt // 2)
    return us_out, ud_out


# speedup vs baseline: 7.9299x; 1.1031x over previous
import functools

import jax
import jax.numpy as jnp
from jax import lax
from jax.experimental import pallas as pl
from jax.experimental.pallas import tpu as pltpu
from jax.experimental.pallas import tpu_sc as plsc

_F32 = jnp.float32


def _wt(w, b):
    return w.T, b.reshape(1, -1)


def _mmg(x, w_t, b):
    bs, n, c = x.shape
    o = w_t.shape[1]

    def body(x_ref, w_ref, b_ref, o_ref):
        o_ref[0] = jax.nn.gelu(
            jnp.dot(x_ref[0], w_ref[...], preferred_element_type=_F32)
            + b_ref[...])

    return pl.pallas_call(
        body,
        grid=(bs,),
        in_specs=[
            pl.BlockSpec((1, n, c), lambda i: (i, 0, 0)),
            pl.BlockSpec((c, o), lambda i: (0, 0)),
            pl.BlockSpec((1, o), lambda i: (0, 0)),
        ],
        out_specs=pl.BlockSpec((1, n, o), lambda i: (i, 0, 0)),
        out_shape=jax.ShapeDtypeStruct((bs, n, o), _F32),
    )(x, w_t, b)


_BF16 = jnp.bfloat16


def _split3(xf):
    hi = xf.astype(_BF16)
    r1 = xf - hi.astype(_F32)
    mid = r1.astype(_BF16)
    lo = (r1 - mid.astype(_F32)).astype(_BF16)
    return hi, mid, lo


def _exact_gather(oh_bf, parts):
    hi, mid, lo = parts
    nb = jnp.dot(oh_bf, hi, preferred_element_type=_F32)
    nb = nb + jnp.dot(oh_bf, mid, preferred_element_type=_F32)
    return nb + jnp.dot(oh_bf, lo, preferred_element_type=_F32)


def _edge_gcn(x, sq_row, sq_col, w1_t, b1, w2_t, b2, k, tile):
    bs, n, c = x.shape
    p = w1_t.shape[1]
    q = w2_t.shape[1]

    def body(xt_ref, xf_ref, sqr_ref, sqc_ref, w1_ref, b1_ref, w2_ref,
             b2_ref, o_ref):
        xt = xt_ref[0]
        xf = xf_ref[0]
        parts = _split3(xf)
        g = lax.dot_general(xt, xf, (((1,), (1,)), ((), ())),
                            preferred_element_type=_F32)
        dist = (sqr_ref[0] - 2.0 * g) + sqc_ref[0]
        iota = lax.broadcasted_iota(jnp.int32, (tile, n), 1)
        acc = jnp.full((tile, q), -jnp.inf, dtype=_F32)
        for _ in range(k):
            m = jnp.min(dist, axis=1, keepdims=True)
            idxm = jnp.min(jnp.where(dist <= m, iota, n), axis=1,
                           keepdims=True)
            oh = iota == idxm
            nb = _exact_gather(oh.astype(_BF16), parts)
            feat = jnp.concatenate([nb - xt, xt], axis=1)
            h = jax.nn.gelu(
                jnp.dot(feat, w1_ref[...], preferred_element_type=_F32)
                + b1_ref[...])
            gl = jax.nn.gelu(jnp.dot(h, w2_ref[...],
                                     preferred_element_type=_F32)
                             + b2_ref[...])
            acc = jnp.maximum(acc, gl)
            dist = jnp.where(oh, jnp.inf, dist)
        o_ref[0] = acc

    return pl.pallas_call(
        body,
        grid=(bs, n // tile),
        in_specs=[
            pl.BlockSpec((1, tile, c), lambda i, j: (i, j, 0)),
            pl.BlockSpec((1, n, c), lambda i, j: (i, 0, 0)),
            pl.BlockSpec((1, tile, 1), lambda i, j: (i, j, 0)),
            pl.BlockSpec((1, 1, n), lambda i, j: (i, 0, 0)),
            pl.BlockSpec((2 * c, p), lambda i, j: (0, 0)),
            pl.BlockSpec((1, p), lambda i, j: (0, 0)),
            pl.BlockSpec((p, q), lambda i, j: (0, 0)),
            pl.BlockSpec((1, q), lambda i, j: (0, 0)),
        ],
        out_specs=pl.BlockSpec((1, tile, q), lambda i, j: (i, j, 0)),
        out_shape=jax.ShapeDtypeStruct((bs, n, q), _F32),
    )(x, x, sq_row, sq_col, w1_t, b1, w2_t, b2)


def _knn_idx(x, sq_row, sq_col, k, tile):
    bs, n, c = x.shape

    def body(xt_ref, xf_ref, sqr_ref, sqc_ref, o_ref):
        g = lax.dot_general(xt_ref[0], xf_ref[0], (((1,), (1,)), ((), ())),
                            preferred_element_type=_F32)
        dist = (sqr_ref[0] - 2.0 * g) + sqc_ref[0]
        iota = lax.broadcasted_iota(jnp.int32, (tile, n), 1)
        base = pl.program_id(0) * n
        cols = []
        for _ in range(k):
            m = jnp.min(dist, axis=1, keepdims=True)
            idxm = jnp.min(jnp.where(dist <= m, iota, n), axis=1,
                           keepdims=True)
            cols.append(idxm + base)
            dist = jnp.where(iota == idxm, jnp.inf, dist)
        o_ref[0] = jnp.concatenate(cols, axis=1)

    return pl.pallas_call(
        body,
        grid=(bs, n // tile),
        in_specs=[
            pl.BlockSpec((1, tile, c), lambda i, j: (i, j, 0)),
            pl.BlockSpec((1, n, c), lambda i, j: (i, 0, 0)),
            pl.BlockSpec((1, tile, 1), lambda i, j: (i, j, 0)),
            pl.BlockSpec((1, 1, n), lambda i, j: (i, 0, 0)),
        ],
        out_specs=pl.BlockSpec((1, tile, k), lambda i, j: (i, j, 0)),
        out_shape=jax.ShapeDtypeStruct((bs, n, k), jnp.int32),
    )(x, x, sq_row, sq_col)


def _sc_gather(table, idx):
    v, d = table.shape
    b = idx.shape[0]
    info = plsc.get_sparse_core_info()
    nw = info.num_cores * info.num_subcores
    b_per_w = b // nw
    chunk = 320
    mesh = plsc.VectorSubcoreMesh(core_axis_name="c", subcore_axis_name="s")

    @functools.partial(
        pl.kernel, mesh=mesh,
        out_type=jax.ShapeDtypeStruct((b, d), _F32),
        scratch_types=[
            pltpu.VMEM((chunk,), jnp.int32),
            pltpu.VMEM((chunk, d), _F32),
            pltpu.SemaphoreType.DMA,
        ],
    )
    def k(table_hbm, idx_hbm, out_hbm, idx_v, rows_v, sem):
        wid = lax.axis_index("s") * info.num_cores + lax.axis_index("c")
        base = wid * b_per_w
        for ci in range(b_per_w // chunk):
            off = base + ci * chunk
            pltpu.sync_copy(idx_hbm.at[pl.ds(off, chunk)], idx_v)
            pltpu.async_copy(table_hbm.at[idx_v], rows_v, sem).wait()
            pltpu.sync_copy(rows_v, out_hbm.at[pl.ds(off, chunk)])

    return k(table, idx)


def _edge_mlp(x, nb, w1_t, b1, w2_t, b2, k, tile):
    bs, n, c = x.shape
    p = w1_t.shape[1]
    q = w2_t.shape[1]

    c_pad = nb.shape[2]

    def body(xt_ref, nb_ref, w1_ref, b1_ref, w2_ref, b2_ref, o_ref):
        xt = xt_ref[0]
        nb3 = nb_ref[0][:, :c].reshape(tile, k, c)
        diff = (nb3 - xt[:, None, :]).reshape(tile * k, c)
        cen = jnp.broadcast_to(xt[:, None, :],
                               (tile, k, c)).reshape(tile * k, c)
        feat = jnp.concatenate([diff, cen], axis=1)
        h = jax.nn.gelu(
            jnp.dot(feat, w1_ref[...], preferred_element_type=_F32)
            + b1_ref[...])
        gl = jax.nn.gelu(jnp.dot(h, w2_ref[...],
                                 preferred_element_type=_F32) + b2_ref[...])
        o_ref[0] = jnp.max(gl.reshape(tile, k, q), axis=1)

    return pl.pallas_call(
        body,
        grid=(bs, n // tile),
        in_specs=[
            pl.BlockSpec((1, tile, c), lambda i, j: (i, j, 0)),
            pl.BlockSpec((1, tile * k, c_pad), lambda i, j: (i, j, 0)),
            pl.BlockSpec((2 * c, p), lambda i, j: (0, 0)),
            pl.BlockSpec((1, p), lambda i, j: (0, 0)),
            pl.BlockSpec((p, q), lambda i, j: (0, 0)),
            pl.BlockSpec((1, q), lambda i, j: (0, 0)),
        ],
        out_specs=pl.BlockSpec((1, tile, q), lambda i, j: (i, j, 0)),
        out_shape=jax.ShapeDtypeStruct((bs, n, q), _F32),
    )(x, nb, w1_t, b1, w2_t, b2)


def _edge_gcn_fast(x, sq_row, sq_col, wa_t, wcm_t, b1, w2_t, b2, k, tile):
    bs, n, c = x.shape
    p = wa_t.shape[1]
    q = w2_t.shape[1]

    def body(xt_ref, xf_ref, sqr_ref, sqc_ref, wa_ref, wc_ref, b1_ref,
             w2_ref, b2_ref, o_ref):
        xt = xt_ref[0]
        xf = xf_ref[0]
        g = lax.dot_general(xt, xf, (((1,), (1,)), ((), ())),
                            preferred_element_type=_F32)
        dist = (sqr_ref[0] - 2.0 * g) + sqc_ref[0]
        a_full = jnp.dot(xf, wa_ref[...],
                         preferred_element_type=_F32).astype(_BF16)
        b_tile = (jnp.dot(xt, wc_ref[...], preferred_element_type=_F32)
                  + b1_ref[...])
        iota = lax.broadcasted_iota(jnp.int32, (tile, n), 1)
        acc = jnp.full((tile, q), -jnp.inf, dtype=_F32)
        for _ in range(k):
            m = jnp.min(dist, axis=1, keepdims=True)
            idxm = jnp.min(jnp.where(dist <= m, iota, n), axis=1,
                           keepdims=True)
            oh = iota == idxm
            nb_a = jnp.dot(oh.astype(_BF16), a_full,
                           preferred_element_type=_F32)
            h = jax.nn.gelu(nb_a + b_tile)
            gl = jax.nn.gelu(jnp.dot(h, w2_ref[...],
                                     preferred_element_type=_F32)
                             + b2_ref[...])
            acc = jnp.maximum(acc, gl)
            dist = jnp.where(oh, jnp.inf, dist)
        o_ref[0] = acc

    return pl.pallas_call(
        body,
        grid=(bs, n // tile),
        in_specs=[
            pl.BlockSpec((1, tile, c), lambda i, j: (i, j, 0)),
            pl.BlockSpec((1, n, c), lambda i, j: (i, 0, 0)),
            pl.BlockSpec((1, tile, 1), lambda i, j: (i, j, 0)),
            pl.BlockSpec((1, 1, n), lambda i, j: (i, 0, 0)),
            pl.BlockSpec((c, p), lambda i, j: (0, 0)),
            pl.BlockSpec((c, p), lambda i, j: (0, 0)),
            pl.BlockSpec((1, p), lambda i, j: (0, 0)),
            pl.BlockSpec((p, q), lambda i, j: (0, 0)),
            pl.BlockSpec((1, q), lambda i, j: (0, 0)),
        ],
        out_specs=pl.BlockSpec((1, tile, q), lambda i, j: (i, j, 0)),
        out_shape=jax.ShapeDtypeStruct((bs, n, q), _F32),
    )(x, x, sq_row, sq_col, wa_t, wcm_t, b1, w2_t, b2)


def _stage1(d5, s, wt_cat, bt, w_d2s_t, b_d2s, n_stk, n_pnt):
    bs, n, c5 = d5.shape
    cd = wt_cat.shape[1]
    cs = s.shape[2]
    u = w_d2s_t.shape[1]

    def body(d5_ref, s_ref, wt_ref, bt_ref, wd2s_ref, bd_ref, us_ref):
        t = jax.nn.gelu(
            jnp.dot(d5_ref[0], wt_ref[...], preferred_element_type=_F32)
            + bt_ref[...])
        spdn = jnp.max(t.reshape(n_stk, n_pnt, cd), axis=1)
        cat = jnp.concatenate([s_ref[0], spdn], axis=1)
        us_ref[0] = jax.nn.gelu(
            jnp.dot(cat, wd2s_ref[...], preferred_element_type=_F32)
            + bd_ref[...])

    return pl.pallas_call(
        body,
        grid=(bs,),
        in_specs=[
            pl.BlockSpec((1, n, c5), lambda i: (i, 0, 0)),
            pl.BlockSpec((1, n_stk, cs), lambda i: (i, 0, 0)),
            pl.BlockSpec((c5, cd), lambda i: (0, 0)),
            pl.BlockSpec((1, cd), lambda i: (0, 0)),
            pl.BlockSpec((cs + cd, u), lambda i: (0, 0)),
            pl.BlockSpec((1, u), lambda i: (0, 0)),
        ],
        out_specs=pl.BlockSpec((1, n_stk, u), lambda i: (i, 0, 0)),
        out_shape=jax.ShapeDtypeStruct((bs, n_stk, u), _F32),
    )(d5, s, wt_cat, bt, w_d2s_t, b_d2s)


def _sparse_gcn(x, sq_row, sq_col, wts, k):
    bs, n, c = x.shape
    (w11, b11, w12, b12, w21, b21, w22, b22, w31, b31, w32, b32) = wts
    q_out = w32.shape[1]

    def gcn_block(xv, sq_t, sq_f, w1_ref, b1_ref, w2_ref, b2_ref):
        parts = _split3(xv)
        if sq_t is None:
            sq_t = jnp.sum(xv * xv, axis=1, keepdims=True)
            ones_c = jnp.ones((1, xv.shape[1]), dtype=_F32)
            y = xv * xv
            yh = y.astype(_BF16)
            r1 = y - yh.astype(_F32)
            ym = r1.astype(_BF16)
            yl = (r1 - ym.astype(_F32)).astype(_BF16)
            ones_b = ones_c.astype(_BF16)
            sq_f = (lax.dot_general(ones_b, yh, (((1,), (1,)), ((), ())),
                                    preferred_element_type=_F32)
                    + lax.dot_general(ones_b, ym, (((1,), (1,)), ((), ())),
                                      preferred_element_type=_F32)
                    + lax.dot_general(ones_b, yl, (((1,), (1,)), ((), ())),
                                      preferred_element_type=_F32))
        g = lax.dot_general(xv, xv, (((1,), (1,)), ((), ())),
                            preferred_element_type=_F32)
        dist = (sq_t - 2.0 * g) + sq_f
        iota = lax.broadcasted_iota(jnp.int32, dist.shape, 1)
        q = w2_ref.shape[1]
        acc = jnp.full((xv.shape[0], q), -jnp.inf, dtype=_F32)
        for _ in range(k):
            m = jnp.min(dist, axis=1, keepdims=True)
            idxm = jnp.min(jnp.where(dist <= m, iota, dist.shape[1]), axis=1,
                           keepdims=True)
            oh = iota == idxm
            nb = _exact_gather(oh.astype(_BF16), parts)
            feat = jnp.concatenate([nb - xv, xv], axis=1)
            h = jax.nn.gelu(
                jnp.dot(feat, w1_ref[...], preferred_element_type=_F32)
                + b1_ref[...])
            gl = jax.nn.gelu(jnp.dot(h, w2_ref[...],
                                     preferred_element_type=_F32)
                             + b2_ref[...])
            acc = jnp.maximum(acc, gl)
            dist = jnp.where(oh, jnp.inf, dist)
        return acc

    def body(x_ref, sqr_ref, sqc_ref, w11_r, b11_r, w12_r, b12_r, w21_r,
             b21_r, w22_r, b22_r, w31_r, b31_r, w32_r, b32_r, o_ref):
        xv = x_ref[0]
        x1 = gcn_block(xv, sqr_ref[0], sqc_ref[0], w11_r, b11_r, w12_r, b12_r)
        x2 = gcn_block(x1, None, None, w21_r, b21_r, w22_r, b22_r)
        xc = jnp.concatenate([x1, x2], axis=1)
        y = jax.nn.gelu(
            jnp.dot(xc, w31_r[...], preferred_element_type=_F32)
            + b31_r[...])
        o_ref[0] = jax.nn.gelu(
            jnp.dot(y, w32_r[...], preferred_element_type=_F32) + b32_r[...])

    w_specs = [pl.BlockSpec(w.shape, lambda i: (0, 0)) for w in wts]
    return pl.pallas_call(
        body,
        grid=(bs,),
        in_specs=[pl.BlockSpec((1, n, c), lambda i: (i, 0, 0)),
                  pl.BlockSpec((1, n, 1), lambda i: (i, 0, 0)),
                  pl.BlockSpec((1, 1, n), lambda i: (i, 0, 0))] + w_specs,
        out_specs=pl.BlockSpec((1, n, q_out), lambda i: (i, 0, 0)),
        out_shape=jax.ShapeDtypeStruct((bs, n, q_out), _F32),
    )(x, sq_row, sq_col, *wts)


def kernel(sparse_fea, dense_fea, params):
    bs, c_dn, n_stk, n_pnt = dense_fea.shape
    c_sp = sparse_fea.shape[1]
    n = n_stk * n_pnt

    s_pm = jnp.transpose(sparse_fea, (0, 2, 1))
    d_pm = jnp.transpose(dense_fea.reshape(bs, c_dn, n), (0, 2, 1))

    d_pad = jnp.pad(d_pm, ((0, 0), (2, 2), (0, 0)), mode='edge')
    d5 = jnp.concatenate([d_pad[:, j:j + n, :] for j in range(5)], axis=2)
    wt_cat = jnp.transpose(params['d2s_tconv_w'], (2, 1, 0)).reshape(-1, c_dn)
    bt = params['d2s_tconv_b'].reshape(1, -1)
    w_d2s, b_d2s = params['d2s_mlp'][0]
    w_s2d, b_s2d = params['s2d_mlp'][0]
    u_s = _stage1(d5, s_pm, wt_cat, bt, w_d2s.T, b_d2s.reshape(1, -1),
                  n_stk, n_pnt)
    rep = jnp.broadcast_to(s_pm[:, :, None, :],
                           (bs, n_stk, n_pnt, c_sp)).reshape(bs, n, c_sp)
    cat_d = jnp.concatenate([d_pm, rep], axis=2)
    u_d = _mmg(cat_d, w_s2d.T, b_s2d.reshape(1, -1))

    def edge_prep(layers):
        (w1, b1), (w2, b2) = layers
        return _wt(w1, b1) + _wt(w2, b2)

    sp = params['sp']
    sp_wts = edge_prep(sp['c1']) + edge_prep(sp['c2']) + edge_prep(sp['c3'])
    sq_us = jnp.sum(u_s * u_s, axis=-1)
    us = _sparse_gcn(u_s, sq_us[:, :, None], sq_us[:, None, :], sp_wts, 2)
    us_out = jnp.transpose(us, (0, 2, 1))

    dn = params['dn']
    sq_ud = jnp.sum(u_d * u_d, axis=-1)
    idx1 = _knn_idx(u_d, sq_ud[:, :, None], sq_ud[:, None, :], 10, 256)
    table = jnp.pad(u_d.reshape(bs * n, -1), ((0, 0), (0, 64)))
    nb1 = _sc_gather(table, idx1.reshape(-1))
    x1 = _edge_mlp(u_d, nb1.reshape(bs, n * 10, -1),
                   *(_wt(*dn['c1'][0]) + _wt(*dn['c1'][1])), 10, 256)
    sq_x1 = jnp.sum(x1 * x1, axis=-1)
    w21, b21 = dn['c2'][0]
    c_x1 = x1.shape[2]
    x2 = _edge_gcn_fast(x1, sq_x1[:, :, None], sq_x1[:, None, :],
                        w21[:, :c_x1].T, (w21[:, c_x1:] - w21[:, :c_x1]).T,
                        b21.reshape(1, -1), *_wt(*dn['c2'][1]), 10, 256)

    (w31d, b31d), (w32d, b32d) = dn['c3']
    xcat = jnp.concatenate([x1, x2], axis=2)
    y = _mmg(xcat, w31d.T, b31d.reshape(1, -1))
    yd = _mmg(y, w32d.T, b32d.reshape(1, -1))

    y_pad = jnp.pad(yd, ((0, 0), (2, 2), (0, 0)), mode='edge')
    cat6 = jnp.concatenate([y_pad[:, j:j + n - 1:2, :] for j in range(6)],
                           axis=2)
    w_ds = jnp.transpose(params['ds_conv_w'], (2, 1, 0)).reshape(-1, 128)
    ud_pm = _mmg(cat6, w_ds, params['ds_conv_b'].reshape(1, -1))
    ud_out = jnp.transpose(ud_pm, (0, 2, 1)).reshape(bs, -1, n_stk,
                                                     n_pnt // 2)
    return us_out, ud_out


# speedup vs baseline: 7.9739x; 1.0056x over previous
import functools

import jax
import jax.numpy as jnp
from jax import lax
from jax.experimental import pallas as pl
from jax.experimental.pallas import tpu as pltpu
from jax.experimental.pallas import tpu_sc as plsc

_F32 = jnp.float32


def _wt(w, b):
    return w.T, b.reshape(1, -1)


def _mmg(x, w_t, b):
    bs, n, c = x.shape
    o = w_t.shape[1]

    def body(x_ref, w_ref, b_ref, o_ref):
        o_ref[0] = jax.nn.gelu(
            jnp.dot(x_ref[0], w_ref[...], preferred_element_type=_F32)
            + b_ref[...])

    return pl.pallas_call(
        body,
        grid=(bs,),
        in_specs=[
            pl.BlockSpec((1, n, c), lambda i: (i, 0, 0)),
            pl.BlockSpec((c, o), lambda i: (0, 0)),
            pl.BlockSpec((1, o), lambda i: (0, 0)),
        ],
        out_specs=pl.BlockSpec((1, n, o), lambda i: (i, 0, 0)),
        out_shape=jax.ShapeDtypeStruct((bs, n, o), _F32),
    )(x, w_t, b)


_BF16 = jnp.bfloat16


def _split3(xf):
    hi = xf.astype(_BF16)
    r1 = xf - hi.astype(_F32)
    mid = r1.astype(_BF16)
    lo = (r1 - mid.astype(_F32)).astype(_BF16)
    return hi, mid, lo


def _exact_gather(oh_bf, parts):
    hi, mid, lo = parts
    nb = jnp.dot(oh_bf, hi, preferred_element_type=_F32)
    nb = nb + jnp.dot(oh_bf, mid, preferred_element_type=_F32)
    return nb + jnp.dot(oh_bf, lo, preferred_element_type=_F32)


def _edge_gcn(x, sq_row, sq_col, w1_t, b1, w2_t, b2, k, tile):
    bs, n, c = x.shape
    p = w1_t.shape[1]
    q = w2_t.shape[1]

    def body(xt_ref, xf_ref, sqr_ref, sqc_ref, w1_ref, b1_ref, w2_ref,
             b2_ref, o_ref):
        xt = xt_ref[0]
        xf = xf_ref[0]
        parts = _split3(xf)
        g = lax.dot_general(xt, xf, (((1,), (1,)), ((), ())),
                            preferred_element_type=_F32)
        dist = (sqr_ref[0] - 2.0 * g) + sqc_ref[0]
        iota = lax.broadcasted_iota(jnp.int32, (tile, n), 1)
        acc = jnp.full((tile, q), -jnp.inf, dtype=_F32)
        for _ in range(k):
            m = jnp.min(dist, axis=1, keepdims=True)
            idxm = jnp.min(jnp.where(dist <= m, iota, n), axis=1,
                           keepdims=True)
            oh = iota == idxm
            nb = _exact_gather(oh.astype(_BF16), parts)
            feat = jnp.concatenate([nb - xt, xt], axis=1)
            h = jax.nn.gelu(
                jnp.dot(feat, w1_ref[...], preferred_element_type=_F32)
                + b1_ref[...])
            gl = jax.nn.gelu(jnp.dot(h, w2_ref[...],
                                     preferred_element_type=_F32)
                             + b2_ref[...])
            acc = jnp.maximum(acc, gl)
            dist = jnp.where(oh, jnp.inf, dist)
        o_ref[0] = acc

    return pl.pallas_call(
        body,
        grid=(bs, n // tile),
        in_specs=[
            pl.BlockSpec((1, tile, c), lambda i, j: (i, j, 0)),
            pl.BlockSpec((1, n, c), lambda i, j: (i, 0, 0)),
            pl.BlockSpec((1, tile, 1), lambda i, j: (i, j, 0)),
            pl.BlockSpec((1, 1, n), lambda i, j: (i, 0, 0)),
            pl.BlockSpec((2 * c, p), lambda i, j: (0, 0)),
            pl.BlockSpec((1, p), lambda i, j: (0, 0)),
            pl.BlockSpec((p, q), lambda i, j: (0, 0)),
            pl.BlockSpec((1, q), lambda i, j: (0, 0)),
        ],
        out_specs=pl.BlockSpec((1, tile, q), lambda i, j: (i, j, 0)),
        out_shape=jax.ShapeDtypeStruct((bs, n, q), _F32),
    )(x, x, sq_row, sq_col, w1_t, b1, w2_t, b2)


def _knn_idx(x, sq_row, sq_col, k, tile):
    bs, n, c = x.shape

    def body(xt_ref, xf_ref, sqr_ref, sqc_ref, o_ref):
        g = lax.dot_general(xt_ref[0], xf_ref[0], (((1,), (1,)), ((), ())),
                            preferred_element_type=_F32)
        dist = (sqr_ref[0] - 2.0 * g) + sqc_ref[0]
        iota = lax.broadcasted_iota(jnp.int32, (tile, n), 1)
        base = pl.program_id(0) * n
        cols = []
        for _ in range(k):
            m = jnp.min(dist, axis=1, keepdims=True)
            idxm = jnp.min(jnp.where(dist <= m, iota, n), axis=1,
                           keepdims=True)
            cols.append(idxm + base)
            dist = jnp.where(iota == idxm, jnp.inf, dist)
        o_ref[0] = jnp.concatenate(cols, axis=1)

    return pl.pallas_call(
        body,
        grid=(bs, n // tile),
        in_specs=[
            pl.BlockSpec((1, tile, c), lambda i, j: (i, j, 0)),
            pl.BlockSpec((1, n, c), lambda i, j: (i, 0, 0)),
            pl.BlockSpec((1, tile, 1), lambda i, j: (i, j, 0)),
            pl.BlockSpec((1, 1, n), lambda i, j: (i, 0, 0)),
        ],
        out_specs=pl.BlockSpec((1, tile, k), lambda i, j: (i, j, 0)),
        out_shape=jax.ShapeDtypeStruct((bs, n, k), jnp.int32),
    )(x, x, sq_row, sq_col)


def _sc_gather(table, idx):
    v, d = table.shape
    b = idx.shape[0]
    info = plsc.get_sparse_core_info()
    nw = info.num_cores * info.num_subcores
    b_per_w = b // nw
    chunk = 160
    mesh = plsc.VectorSubcoreMesh(core_axis_name="c", subcore_axis_name="s")

    n_chunk = b_per_w // chunk

    @functools.partial(
        pl.kernel, mesh=mesh,
        out_type=jax.ShapeDtypeStruct((b, d), _F32),
        scratch_types=[
            pltpu.VMEM((chunk,), jnp.int32),
            pltpu.VMEM((chunk, d), _F32),
            pltpu.VMEM((chunk, d), _F32),
            pltpu.SemaphoreType.DMA,
            pltpu.SemaphoreType.DMA,
        ],
    )
    def k(table_hbm, idx_hbm, out_hbm, idx_v, rows_a, rows_b, sem_a, sem_b):
        wid = lax.axis_index("s") * info.num_cores + lax.axis_index("c")
        base = wid * b_per_w
        bufs = [(rows_a, sem_a), (rows_b, sem_b)]
        pltpu.sync_copy(idx_hbm.at[pl.ds(base, chunk)], idx_v)
        cp = pltpu.async_copy(table_hbm.at[idx_v], rows_a, sem_a)
        for ci in range(n_chunk):
            rows, sem = bufs[ci % 2]
            cp.wait()
            if ci + 1 < n_chunk:
                nrows, nsem = bufs[(ci + 1) % 2]
                pltpu.sync_copy(
                    idx_hbm.at[pl.ds(base + (ci + 1) * chunk, chunk)], idx_v)
                cp = pltpu.async_copy(table_hbm.at[idx_v], nrows, nsem)
            pltpu.sync_copy(rows, out_hbm.at[pl.ds(base + ci * chunk, chunk)])

    return k(table, idx)


def _edge_mlp(x, nb, w1_t, b1, w2_t, b2, k, tile):
    bs, n, c = x.shape
    p = w1_t.shape[1]
    q = w2_t.shape[1]

    c_pad = nb.shape[2]

    def body(xt_ref, nb_ref, w1_ref, b1_ref, w2_ref, b2_ref, o_ref):
        xt = xt_ref[0]
        nb3 = nb_ref[0][:, :c].reshape(tile, k, c)
        diff = (nb3 - xt[:, None, :]).reshape(tile * k, c)
        cen = jnp.broadcast_to(xt[:, None, :],
                               (tile, k, c)).reshape(tile * k, c)
        feat = jnp.concatenate([diff, cen], axis=1)
        h = jax.nn.gelu(
            jnp.dot(feat, w1_ref[...], preferred_element_type=_F32)
            + b1_ref[...])
        gl = jax.nn.gelu(jnp.dot(h, w2_ref[...],
                                 preferred_element_type=_F32) + b2_ref[...])
        o_ref[0] = jnp.max(gl.reshape(tile, k, q), axis=1)

    return pl.pallas_call(
        body,
        grid=(bs, n // tile),
        in_specs=[
            pl.BlockSpec((1, tile, c), lambda i, j: (i, j, 0)),
            pl.BlockSpec((1, tile * k, c_pad), lambda i, j: (i, j, 0)),
            pl.BlockSpec((2 * c, p), lambda i, j: (0, 0)),
            pl.BlockSpec((1, p), lambda i, j: (0, 0)),
            pl.BlockSpec((p, q), lambda i, j: (0, 0)),
            pl.BlockSpec((1, q), lambda i, j: (0, 0)),
        ],
        out_specs=pl.BlockSpec((1, tile, q), lambda i, j: (i, j, 0)),
        out_shape=jax.ShapeDtypeStruct((bs, n, q), _F32),
    )(x, nb, w1_t, b1, w2_t, b2)


def _edge_gcn_fast(x, sq_row, sq_col, wa_t, wcm_t, b1, w2_t, b2, k, tile):
    bs, n, c = x.shape
    p = wa_t.shape[1]
    q = w2_t.shape[1]

    def body(xt_ref, xf_ref, sqr_ref, sqc_ref, wa_ref, wc_ref, b1_ref,
             w2_ref, b2_ref, o_ref):
        xt = xt_ref[0]
        xf = xf_ref[0]
        g = lax.dot_general(xt, xf, (((1,), (1,)), ((), ())),
                            preferred_element_type=_F32)
        dist = (sqr_ref[0] - 2.0 * g) + sqc_ref[0]
        a_full = jnp.dot(xf, wa_ref[...],
                         preferred_element_type=_F32).astype(_BF16)
        b_tile = (jnp.dot(xt, wc_ref[...], preferred_element_type=_F32)
                  + b1_ref[...])
        iota = lax.broadcasted_iota(jnp.int32, (tile, n), 1)
        acc = jnp.full((tile, q), -jnp.inf, dtype=_F32)
        for _ in range(k):
            m = jnp.min(dist, axis=1, keepdims=True)
            idxm = jnp.min(jnp.where(dist <= m, iota, n), axis=1,
                           keepdims=True)
            oh = iota == idxm
            nb_a = jnp.dot(oh.astype(_BF16), a_full,
                           preferred_element_type=_F32)
            h = jax.nn.gelu(nb_a + b_tile)
            gl = jax.nn.gelu(jnp.dot(h, w2_ref[...],
                                     preferred_element_type=_F32)
                             + b2_ref[...])
            acc = jnp.maximum(acc, gl)
            dist = jnp.where(oh, jnp.inf, dist)
        o_ref[0] = acc

    return pl.pallas_call(
        body,
        grid=(bs, n // tile),
        in_specs=[
            pl.BlockSpec((1, tile, c), lambda i, j: (i, j, 0)),
            pl.BlockSpec((1, n, c), lambda i, j: (i, 0, 0)),
            pl.BlockSpec((1, tile, 1), lambda i, j: (i, j, 0)),
            pl.BlockSpec((1, 1, n), lambda i, j: (i, 0, 0)),
            pl.BlockSpec((c, p), lambda i, j: (0, 0)),
            pl.BlockSpec((c, p), lambda i, j: (0, 0)),
            pl.BlockSpec((1, p), lambda i, j: (0, 0)),
            pl.BlockSpec((p, q), lambda i, j: (0, 0)),
            pl.BlockSpec((1, q), lambda i, j: (0, 0)),
        ],
        out_specs=pl.BlockSpec((1, tile, q), lambda i, j: (i, j, 0)),
        out_shape=jax.ShapeDtypeStruct((bs, n, q), _F32),
    )(x, x, sq_row, sq_col, wa_t, wcm_t, b1, w2_t, b2)


def _stage1(d5, s, wt_cat, bt, w_d2s_t, b_d2s, n_stk, n_pnt):
    bs, n, c5 = d5.shape
    cd = wt_cat.shape[1]
    cs = s.shape[2]
    u = w_d2s_t.shape[1]

    def body(d5_ref, s_ref, wt_ref, bt_ref, wd2s_ref, bd_ref, us_ref):
        t = jax.nn.gelu(
            jnp.dot(d5_ref[0], wt_ref[...], preferred_element_type=_F32)
            + bt_ref[...])
        spdn = jnp.max(t.reshape(n_stk, n_pnt, cd), axis=1)
        cat = jnp.concatenate([s_ref[0], spdn], axis=1)
        us_ref[0] = jax.nn.gelu(
            jnp.dot(cat, wd2s_ref[...], preferred_element_type=_F32)
            + bd_ref[...])

    return pl.pallas_call(
        body,
        grid=(bs,),
        in_specs=[
            pl.BlockSpec((1, n, c5), lambda i: (i, 0, 0)),
            pl.BlockSpec((1, n_stk, cs), lambda i: (i, 0, 0)),
            pl.BlockSpec((c5, cd), lambda i: (0, 0)),
            pl.BlockSpec((1, cd), lambda i: (0, 0)),
            pl.BlockSpec((cs + cd, u), lambda i: (0, 0)),
            pl.BlockSpec((1, u), lambda i: (0, 0)),
        ],
        out_specs=pl.BlockSpec((1, n_stk, u), lambda i: (i, 0, 0)),
        out_shape=jax.ShapeDtypeStruct((bs, n_stk, u), _F32),
    )(d5, s, wt_cat, bt, w_d2s_t, b_d2s)


def _sparse_gcn(x, sq_row, sq_col, wts, k):
    bs, n, c = x.shape
    (w11, b11, w12, b12, w21, b21, w22, b22, w31, b31, w32, b32) = wts
    q_out = w32.shape[1]

    def gcn_block(xv, sq_t, sq_f, w1_ref, b1_ref, w2_ref, b2_ref):
        parts = _split3(xv)
        if sq_t is None:
            sq_t = jnp.sum(xv * xv, axis=1, keepdims=True)
            ones_c = jnp.ones((1, xv.shape[1]), dtype=_F32)
            y = xv * xv
            yh = y.astype(_BF16)
            r1 = y - yh.astype(_F32)
            ym = r1.astype(_BF16)
            yl = (r1 - ym.astype(_F32)).astype(_BF16)
            ones_b = ones_c.astype(_BF16)
            sq_f = (lax.dot_general(ones_b, yh, (((1,), (1,)), ((), ())),
                                    preferred_element_type=_F32)
                    + lax.dot_general(ones_b, ym, (((1,), (1,)), ((), ())),
                                      preferred_element_type=_F32)
                    + lax.dot_general(ones_b, yl, (((1,), (1,)), ((), ())),
                                      preferred_element_type=_F32))
        g = lax.dot_general(xv, xv, (((1,), (1,)), ((), ())),
                            preferred_element_type=_F32)
        dist = (sq_t - 2.0 * g) + sq_f
        iota = lax.broadcasted_iota(jnp.int32, dist.shape, 1)
        q = w2_ref.shape[1]
        acc = jnp.full((xv.shape[0], q), -jnp.inf, dtype=_F32)
        for _ in range(k):
            m = jnp.min(dist, axis=1, keepdims=True)
            idxm = jnp.min(jnp.where(dist <= m, iota, dist.shape[1]), axis=1,
                           keepdims=True)
            oh = iota == idxm
            nb = _exact_gather(oh.astype(_BF16), parts)
            feat = jnp.concatenate([nb - xv, xv], axis=1)
            h = jax.nn.gelu(
                jnp.dot(feat, w1_ref[...], preferred_element_type=_F32)
                + b1_ref[...])
            gl = jax.nn.gelu(jnp.dot(h, w2_ref[...],
                                     preferred_element_type=_F32)
                             + b2_ref[...])
            acc = jnp.maximum(acc, gl)
            dist = jnp.where(oh, jnp.inf, dist)
        return acc

    def body(x_ref, sqr_ref, sqc_ref, w11_r, b11_r, w12_r, b12_r, w21_r,
             b21_r, w22_r, b22_r, w31_r, b31_r, w32_r, b32_r, o_ref):
        xv = x_ref[0]
        x1 = gcn_block(xv, sqr_ref[0], sqc_ref[0], w11_r, b11_r, w12_r, b12_r)
        x2 = gcn_block(x1, None, None, w21_r, b21_r, w22_r, b22_r)
        xc = jnp.concatenate([x1, x2], axis=1)
        y = jax.nn.gelu(
            jnp.dot(xc, w31_r[...], preferred_element_type=_F32)
            + b31_r[...])
        o_ref[0] = jax.nn.gelu(
            jnp.dot(y, w32_r[...], preferred_element_type=_F32) + b32_r[...])

    w_specs = [pl.BlockSpec(w.shape, lambda i: (0, 0)) for w in wts]
    return pl.pallas_call(
        body,
        grid=(bs,),
        in_specs=[pl.BlockSpec((1, n, c), lambda i: (i, 0, 0)),
                  pl.BlockSpec((1, n, 1), lambda i: (i, 0, 0)),
                  pl.BlockSpec((1, 1, n), lambda i: (i, 0, 0))] + w_specs,
        out_specs=pl.BlockSpec((1, n, q_out), lambda i: (i, 0, 0)),
        out_shape=jax.ShapeDtypeStruct((bs, n, q_out), _F32),
    )(x, sq_row, sq_col, *wts)


def kernel(sparse_fea, dense_fea, params):
    bs, c_dn, n_stk, n_pnt = dense_fea.shape
    c_sp = sparse_fea.shape[1]
    n = n_stk * n_pnt

    s_pm = jnp.transpose(sparse_fea, (0, 2, 1))
    d_pm = jnp.transpose(dense_fea.reshape(bs, c_dn, n), (0, 2, 1))

    d_pad = jnp.pad(d_pm, ((0, 0), (2, 2), (0, 0)), mode='edge')
    d5 = jnp.concatenate([d_pad[:, j:j + n, :] for j in range(5)], axis=2)
    wt_cat = jnp.transpose(params['d2s_tconv_w'], (2, 1, 0)).reshape(-1, c_dn)
    bt = params['d2s_tconv_b'].reshape(1, -1)
    w_d2s, b_d2s = params['d2s_mlp'][0]
    w_s2d, b_s2d = params['s2d_mlp'][0]
    u_s = _stage1(d5, s_pm, wt_cat, bt, w_d2s.T, b_d2s.reshape(1, -1),
                  n_stk, n_pnt)
    rep = jnp.broadcast_to(s_pm[:, :, None, :],
                           (bs, n_stk, n_pnt, c_sp)).reshape(bs, n, c_sp)
    cat_d = jnp.concatenate([d_pm, rep], axis=2)
    u_d = _mmg(cat_d, w_s2d.T, b_s2d.reshape(1, -1))

    def edge_prep(layers):
        (w1, b1), (w2, b2) = layers
        return _wt(w1, b1) + _wt(w2, b2)

    sp = params['sp']
    sp_wts = edge_prep(sp['c1']) + edge_prep(sp['c2']) + edge_prep(sp['c3'])
    sq_us = jnp.sum(u_s * u_s, axis=-1)
    us = _sparse_gcn(u_s, sq_us[:, :, None], sq_us[:, None, :], sp_wts, 2)
    us_out = jnp.transpose(us, (0, 2, 1))

    dn = params['dn']
    sq_ud = jnp.sum(u_d * u_d, axis=-1)
    idx1 = _knn_idx(u_d, sq_ud[:, :, None], sq_ud[:, None, :], 10, 256)
    table = jnp.pad(u_d.reshape(bs * n, -1), ((0, 0), (0, 64)))
    nb1 = _sc_gather(table, idx1.reshape(-1))
    x1 = _edge_mlp(u_d, nb1.reshape(bs, n * 10, -1),
                   *(_wt(*dn['c1'][0]) + _wt(*dn['c1'][1])), 10, 256)
    sq_x1 = jnp.sum(x1 * x1, axis=-1)
    w21, b21 = dn['c2'][0]
    c_x1 = x1.shape[2]
    x2 = _edge_gcn_fast(x1, sq_x1[:, :, None], sq_x1[:, None, :],
                        w21[:, :c_x1].T, (w21[:, c_x1:] - w21[:, :c_x1]).T,
                        b21.reshape(1, -1), *_wt(*dn['c2'][1]), 10, 256)

    (w31d, b31d), (w32d, b32d) = dn['c3']
    xcat = jnp.concatenate([x1, x2], axis=2)
    y = _mmg(xcat, w31d.T, b31d.reshape(1, -1))
    yd = _mmg(y, w32d.T, b32d.reshape(1, -1))

    y_pad = jnp.pad(yd, ((0, 0), (2, 2), (0, 0)), mode='edge')
    cat6 = jnp.concatenate([y_pad[:, j:j + n - 1:2, :] for j in range(6)],
                           axis=2)
    w_ds = jnp.transpose(params['ds_conv_w'], (2, 1, 0)).reshape(-1, 128)
    ud_pm = _mmg(cat6, w_ds, params['ds_conv_b'].reshape(1, -1))
    ud_out = jnp.transpose(ud_pm, (0, 2, 1)).reshape(bs, -1, n_stk,
                                                     n_pnt // 2)
    return us_out, ud_out


# speedup vs baseline: 8.5144x; 1.0678x over previous
import functools

import jax
import jax.numpy as jnp
from jax import lax
from jax.experimental import pallas as pl
from jax.experimental.pallas import tpu as pltpu
from jax.experimental.pallas import tpu_sc as plsc

_F32 = jnp.float32


def _wt(w, b):
    return w.T, b.reshape(1, -1)


def _mmg(x, w_t, b):
    bs, n, c = x.shape
    o = w_t.shape[1]

    def body(x_ref, w_ref, b_ref, o_ref):
        o_ref[0] = jax.nn.gelu(
            jnp.dot(x_ref[0], w_ref[...], preferred_element_type=_F32)
            + b_ref[...])

    return pl.pallas_call(
        body,
        grid=(bs,),
        in_specs=[
            pl.BlockSpec((1, n, c), lambda i: (i, 0, 0)),
            pl.BlockSpec((c, o), lambda i: (0, 0)),
            pl.BlockSpec((1, o), lambda i: (0, 0)),
        ],
        out_specs=pl.BlockSpec((1, n, o), lambda i: (i, 0, 0)),
        out_shape=jax.ShapeDtypeStruct((bs, n, o), _F32),
    )(x, w_t, b)


_BF16 = jnp.bfloat16


def _split3(xf):
    hi = xf.astype(_BF16)
    r1 = xf - hi.astype(_F32)
    mid = r1.astype(_BF16)
    lo = (r1 - mid.astype(_F32)).astype(_BF16)
    return hi, mid, lo


def _exact_gather(oh_bf, parts):
    hi, mid, lo = parts
    nb = jnp.dot(oh_bf, hi, preferred_element_type=_F32)
    nb = nb + jnp.dot(oh_bf, mid, preferred_element_type=_F32)
    return nb + jnp.dot(oh_bf, lo, preferred_element_type=_F32)


def _edge_gcn(x, sq_row, sq_col, w1_t, b1, w2_t, b2, k, tile):
    bs, n, c = x.shape
    p = w1_t.shape[1]
    q = w2_t.shape[1]

    def body(xt_ref, xf_ref, sqr_ref, sqc_ref, w1_ref, b1_ref, w2_ref,
             b2_ref, o_ref):
        xt = xt_ref[0]
        xf = xf_ref[0]
        parts = _split3(xf)
        g = lax.dot_general(xt, xf, (((1,), (1,)), ((), ())),
                            preferred_element_type=_F32)
        dist = (sqr_ref[0] - 2.0 * g) + sqc_ref[0]
        iota = lax.broadcasted_iota(jnp.int32, (tile, n), 1)
        acc = jnp.full((tile, q), -jnp.inf, dtype=_F32)
        for _ in range(k):
            idxm = jnp.argmin(dist, axis=1)[:, None]
            oh = iota == idxm
            nb = _exact_gather(oh.astype(_BF16), parts)
            feat = jnp.concatenate([nb - xt, xt], axis=1)
            h = jax.nn.gelu(
                jnp.dot(feat, w1_ref[...], preferred_element_type=_F32)
                + b1_ref[...])
            gl = jax.nn.gelu(jnp.dot(h, w2_ref[...],
                                     preferred_element_type=_F32)
                             + b2_ref[...])
            acc = jnp.maximum(acc, gl)
            dist = jnp.where(oh, jnp.inf, dist)
        o_ref[0] = acc

    return pl.pallas_call(
        body,
        grid=(bs, n // tile),
        in_specs=[
            pl.BlockSpec((1, tile, c), lambda i, j: (i, j, 0)),
            pl.BlockSpec((1, n, c), lambda i, j: (i, 0, 0)),
            pl.BlockSpec((1, tile, 1), lambda i, j: (i, j, 0)),
            pl.BlockSpec((1, 1, n), lambda i, j: (i, 0, 0)),
            pl.BlockSpec((2 * c, p), lambda i, j: (0, 0)),
            pl.BlockSpec((1, p), lambda i, j: (0, 0)),
            pl.BlockSpec((p, q), lambda i, j: (0, 0)),
            pl.BlockSpec((1, q), lambda i, j: (0, 0)),
        ],
        out_specs=pl.BlockSpec((1, tile, q), lambda i, j: (i, j, 0)),
        out_shape=jax.ShapeDtypeStruct((bs, n, q), _F32),
    )(x, x, sq_row, sq_col, w1_t, b1, w2_t, b2)


def _knn_idx(x, sq_row, sq_col, k, tile):
    bs, n, c = x.shape

    def body(xt_ref, xf_ref, sqr_ref, sqc_ref, o_ref):
        g = lax.dot_general(xt_ref[0], xf_ref[0], (((1,), (1,)), ((), ())),
                            preferred_element_type=_F32)
        dist = (sqr_ref[0] - 2.0 * g) + sqc_ref[0]
        iota = lax.broadcasted_iota(jnp.int32, (tile, n), 1)
        base = pl.program_id(0) * n
        cols = []
        for _ in range(k):
            idxm = jnp.argmin(dist, axis=1)[:, None]
            cols.append(idxm + base)
            dist = jnp.where(iota == idxm, jnp.inf, dist)
        o_ref[0] = jnp.concatenate(cols, axis=1)

    return pl.pallas_call(
        body,
        grid=(bs, n // tile),
        in_specs=[
            pl.BlockSpec((1, tile, c), lambda i, j: (i, j, 0)),
            pl.BlockSpec((1, n, c), lambda i, j: (i, 0, 0)),
            pl.BlockSpec((1, tile, 1), lambda i, j: (i, j, 0)),
            pl.BlockSpec((1, 1, n), lambda i, j: (i, 0, 0)),
        ],
        out_specs=pl.BlockSpec((1, tile, k), lambda i, j: (i, j, 0)),
        out_shape=jax.ShapeDtypeStruct((bs, n, k), jnp.int32),
    )(x, x, sq_row, sq_col)


def _sc_gather(table, idx):
    v, d = table.shape
    b = idx.shape[0]
    info = plsc.get_sparse_core_info()
    nw = info.num_cores * info.num_subcores
    b_per_w = b // nw
    chunk = 160
    mesh = plsc.VectorSubcoreMesh(core_axis_name="c", subcore_axis_name="s")

    n_chunk = b_per_w // chunk

    @functools.partial(
        pl.kernel, mesh=mesh,
        out_type=jax.ShapeDtypeStruct((b, d), _F32),
        scratch_types=[
            pltpu.VMEM((chunk,), jnp.int32),
            pltpu.VMEM((chunk, d), _F32),
            pltpu.VMEM((chunk, d), _F32),
            pltpu.SemaphoreType.DMA,
            pltpu.SemaphoreType.DMA,
        ],
    )
    def k(table_hbm, idx_hbm, out_hbm, idx_v, rows_a, rows_b, sem_a, sem_b):
        wid = lax.axis_index("s") * info.num_cores + lax.axis_index("c")
        base = wid * b_per_w
        bufs = [(rows_a, sem_a), (rows_b, sem_b)]
        pltpu.sync_copy(idx_hbm.at[pl.ds(base, chunk)], idx_v)
        cp = pltpu.async_copy(table_hbm.at[idx_v], rows_a, sem_a)
        for ci in range(n_chunk):
            rows, sem = bufs[ci % 2]
            cp.wait()
            if ci + 1 < n_chunk:
                nrows, nsem = bufs[(ci + 1) % 2]
                pltpu.sync_copy(
                    idx_hbm.at[pl.ds(base + (ci + 1) * chunk, chunk)], idx_v)
                cp = pltpu.async_copy(table_hbm.at[idx_v], nrows, nsem)
            pltpu.sync_copy(rows, out_hbm.at[pl.ds(base + ci * chunk, chunk)])

    return k(table, idx)


def _edge_mlp(x, nb, w1_t, b1, w2_t, b2, k, tile):
    bs, n, c = x.shape
    p = w1_t.shape[1]
    q = w2_t.shape[1]

    c_pad = nb.shape[2]

    def body(xt_ref, nb_ref, w1_ref, b1_ref, w2_ref, b2_ref, o_ref):
        xt = xt_ref[0]
        nb3 = nb_ref[0][:, :c].reshape(tile, k, c)
        diff = (nb3 - xt[:, None, :]).reshape(tile * k, c)
        cen = jnp.broadcast_to(xt[:, None, :],
                               (tile, k, c)).reshape(tile * k, c)
        feat = jnp.concatenate([diff, cen], axis=1)
        h = jax.nn.gelu(
            jnp.dot(feat, w1_ref[...], preferred_element_type=_F32)
            + b1_ref[...])
        gl = jax.nn.gelu(jnp.dot(h, w2_ref[...],
                                 preferred_element_type=_F32) + b2_ref[...])
        o_ref[0] = jnp.max(gl.reshape(tile, k, q), axis=1)

    return pl.pallas_call(
        body,
        grid=(bs, n // tile),
        in_specs=[
            pl.BlockSpec((1, tile, c), lambda i, j: (i, j, 0)),
            pl.BlockSpec((1, tile * k, c_pad), lambda i, j: (i, j, 0)),
            pl.BlockSpec((2 * c, p), lambda i, j: (0, 0)),
            pl.BlockSpec((1, p), lambda i, j: (0, 0)),
            pl.BlockSpec((p, q), lambda i, j: (0, 0)),
            pl.BlockSpec((1, q), lambda i, j: (0, 0)),
        ],
        out_specs=pl.BlockSpec((1, tile, q), lambda i, j: (i, j, 0)),
        out_shape=jax.ShapeDtypeStruct((bs, n, q), _F32),
    )(x, nb, w1_t, b1, w2_t, b2)


def _edge_gcn_fast(x, sq_row, sq_col, wa_t, wcm_t, b1, w2_t, b2, k, tile):
    bs, n, c = x.shape
    p = wa_t.shape[1]
    q = w2_t.shape[1]

    def body(xt_ref, xf_ref, sqr_ref, sqc_ref, wa_ref, wc_ref, b1_ref,
             w2_ref, b2_ref, o_ref):
        xt = xt_ref[0]
        xf = xf_ref[0]
        g = lax.dot_general(xt, xf, (((1,), (1,)), ((), ())),
                            preferred_element_type=_F32)
        dist = (sqr_ref[0] - 2.0 * g) + sqc_ref[0]
        a_full = jnp.dot(xf, wa_ref[...],
                         preferred_element_type=_F32).astype(_BF16)
        b_tile = (jnp.dot(xt, wc_ref[...], preferred_element_type=_F32)
                  + b1_ref[...])
        iota = lax.broadcasted_iota(jnp.int32, (tile, n), 1)
        acc = jnp.full((tile, q), -jnp.inf, dtype=_F32)
        for _ in range(k):
            idxm = jnp.argmin(dist, axis=1)[:, None]
            oh = iota == idxm
            nb_a = jnp.dot(oh.astype(_BF16), a_full,
                           preferred_element_type=_F32)
            h = jax.nn.gelu(nb_a + b_tile)
            gl = jax.nn.gelu(jnp.dot(h, w2_ref[...],
                                     preferred_element_type=_F32)
                             + b2_ref[...])
            acc = jnp.maximum(acc, gl)
            dist = jnp.where(oh, jnp.inf, dist)
        o_ref[0] = acc

    return pl.pallas_call(
        body,
        grid=(bs, n // tile),
        in_specs=[
            pl.BlockSpec((1, tile, c), lambda i, j: (i, j, 0)),
            pl.BlockSpec((1, n, c), lambda i, j: (i, 0, 0)),
            pl.BlockSpec((1, tile, 1), lambda i, j: (i, j, 0)),
            pl.BlockSpec((1, 1, n), lambda i, j: (i, 0, 0)),
            pl.BlockSpec((c, p), lambda i, j: (0, 0)),
            pl.BlockSpec((c, p), lambda i, j: (0, 0)),
            pl.BlockSpec((1, p), lambda i, j: (0, 0)),
            pl.BlockSpec((p, q), lambda i, j: (0, 0)),
            pl.BlockSpec((1, q), lambda i, j: (0, 0)),
        ],
        out_specs=pl.BlockSpec((1, tile, q), lambda i, j: (i, j, 0)),
        out_shape=jax.ShapeDtypeStruct((bs, n, q), _F32),
    )(x, x, sq_row, sq_col, wa_t, wcm_t, b1, w2_t, b2)


def _stage1(d5, s, wt_cat, bt, w_d2s_t, b_d2s, n_stk, n_pnt):
    bs, n, c5 = d5.shape
    cd = wt_cat.shape[1]
    cs = s.shape[2]
    u = w_d2s_t.shape[1]

    def body(d5_ref, s_ref, wt_ref, bt_ref, wd2s_ref, bd_ref, us_ref):
        t = jax.nn.gelu(
            jnp.dot(d5_ref[0], wt_ref[...], preferred_element_type=_F32)
            + bt_ref[...])
        spdn = jnp.max(t.reshape(n_stk, n_pnt, cd), axis=1)
        cat = jnp.concatenate([s_ref[0], spdn], axis=1)
        us_ref[0] = jax.nn.gelu(
            jnp.dot(cat, wd2s_ref[...], preferred_element_type=_F32)
            + bd_ref[...])

    return pl.pallas_call(
        body,
        grid=(bs,),
        in_specs=[
            pl.BlockSpec((1, n, c5), lambda i: (i, 0, 0)),
            pl.BlockSpec((1, n_stk, cs), lambda i: (i, 0, 0)),
            pl.BlockSpec((c5, cd), lambda i: (0, 0)),
            pl.BlockSpec((1, cd), lambda i: (0, 0)),
            pl.BlockSpec((cs + cd, u), lambda i: (0, 0)),
            pl.BlockSpec((1, u), lambda i: (0, 0)),
        ],
        out_specs=pl.BlockSpec((1, n_stk, u), lambda i: (i, 0, 0)),
        out_shape=jax.ShapeDtypeStruct((bs, n_stk, u), _F32),
    )(d5, s, wt_cat, bt, w_d2s_t, b_d2s)


def _sparse_gcn(x, sq_row, sq_col, wts, k):
    bs, n, c = x.shape
    (w11, b11, w12, b12, w21, b21, w22, b22, w31, b31, w32, b32) = wts
    q_out = w32.shape[1]

    def gcn_block(xv, sq_t, sq_f, w1_ref, b1_ref, w2_ref, b2_ref):
        parts = _split3(xv)
        if sq_t is None:
            sq_t = jnp.sum(xv * xv, axis=1, keepdims=True)
            ones_c = jnp.ones((1, xv.shape[1]), dtype=_F32)
            y = xv * xv
            yh = y.astype(_BF16)
            r1 = y - yh.astype(_F32)
            ym = r1.astype(_BF16)
            yl = (r1 - ym.astype(_F32)).astype(_BF16)
            ones_b = ones_c.astype(_BF16)
            sq_f = (lax.dot_general(ones_b, yh, (((1,), (1,)), ((), ())),
                                    preferred_element_type=_F32)
                    + lax.dot_general(ones_b, ym, (((1,), (1,)), ((), ())),
                                      preferred_element_type=_F32)
                    + lax.dot_general(ones_b, yl, (((1,), (1,)), ((), ())),
                                      preferred_element_type=_F32))
        g = lax.dot_general(xv, xv, (((1,), (1,)), ((), ())),
                            preferred_element_type=_F32)
        dist = (sq_t - 2.0 * g) + sq_f
        iota = lax.broadcasted_iota(jnp.int32, dist.shape, 1)
        q = w2_ref.shape[1]
        acc = jnp.full((xv.shape[0], q), -jnp.inf, dtype=_F32)
        for _ in range(k):
            idxm = jnp.argmin(dist, axis=1)[:, None]
            oh = iota == idxm
            nb = _exact_gather(oh.astype(_BF16), parts)
            feat = jnp.concatenate([nb - xv, xv], axis=1)
            h = jax.nn.gelu(
                jnp.dot(feat, w1_ref[...], preferred_element_type=_F32)
                + b1_ref[...])
            gl = jax.nn.gelu(jnp.dot(h, w2_ref[...],
                                     preferred_element_type=_F32)
                             + b2_ref[...])
            acc = jnp.maximum(acc, gl)
            dist = jnp.where(oh, jnp.inf, dist)
        return acc

    def body(x_ref, sqr_ref, sqc_ref, w11_r, b11_r, w12_r, b12_r, w21_r,
             b21_r, w22_r, b22_r, w31_r, b31_r, w32_r, b32_r, o_ref):
        xv = x_ref[0]
        x1 = gcn_block(xv, sqr_ref[0], sqc_ref[0], w11_r, b11_r, w12_r, b12_r)
        x2 = gcn_block(x1, None, None, w21_r, b21_r, w22_r, b22_r)
        xc = jnp.concatenate([x1, x2], axis=1)
        y = jax.nn.gelu(
            jnp.dot(xc, w31_r[...], preferred_element_type=_F32)
            + b31_r[...])
        o_ref[0] = jax.nn.gelu(
            jnp.dot(y, w32_r[...], preferred_element_type=_F32) + b32_r[...])

    w_specs = [pl.BlockSpec(w.shape, lambda i: (0, 0)) for w in wts]
    return pl.pallas_call(
        body,
        grid=(bs,),
        in_specs=[pl.BlockSpec((1, n, c), lambda i: (i, 0, 0)),
                  pl.BlockSpec((1, n, 1), lambda i: (i, 0, 0)),
                  pl.BlockSpec((1, 1, n), lambda i: (i, 0, 0))] + w_specs,
        out_specs=pl.BlockSpec((1, n, q_out), lambda i: (i, 0, 0)),
        out_shape=jax.ShapeDtypeStruct((bs, n, q_out), _F32),
    )(x, sq_row, sq_col, *wts)


def kernel(sparse_fea, dense_fea, params):
    bs, c_dn, n_stk, n_pnt = dense_fea.shape
    c_sp = sparse_fea.shape[1]
    n = n_stk * n_pnt

    s_pm = jnp.transpose(sparse_fea, (0, 2, 1))
    d_pm = jnp.transpose(dense_fea.reshape(bs, c_dn, n), (0, 2, 1))

    d_pad = jnp.pad(d_pm, ((0, 0), (2, 2), (0, 0)), mode='edge')
    d5 = jnp.concatenate([d_pad[:, j:j + n, :] for j in range(5)], axis=2)
    wt_cat = jnp.transpose(params['d2s_tconv_w'], (2, 1, 0)).reshape(-1, c_dn)
    bt = params['d2s_tconv_b'].reshape(1, -1)
    w_d2s, b_d2s = params['d2s_mlp'][0]
    w_s2d, b_s2d = params['s2d_mlp'][0]
    u_s = _stage1(d5, s_pm, wt_cat, bt, w_d2s.T, b_d2s.reshape(1, -1),
                  n_stk, n_pnt)
    rep = jnp.broadcast_to(s_pm[:, :, None, :],
                           (bs, n_stk, n_pnt, c_sp)).reshape(bs, n, c_sp)
    cat_d = jnp.concatenate([d_pm, rep], axis=2)
    u_d = _mmg(cat_d, w_s2d.T, b_s2d.reshape(1, -1))

    def edge_prep(layers):
        (w1, b1), (w2, b2) = layers
        return _wt(w1, b1) + _wt(w2, b2)

    sp = params['sp']
    sp_wts = edge_prep(sp['c1']) + edge_prep(sp['c2']) + edge_prep(sp['c3'])
    sq_us = jnp.sum(u_s * u_s, axis=-1)
    us = _sparse_gcn(u_s, sq_us[:, :, None], sq_us[:, None, :], sp_wts, 2)
    us_out = jnp.transpose(us, (0, 2, 1))

    dn = params['dn']
    sq_ud = jnp.sum(u_d * u_d, axis=-1)
    idx1 = _knn_idx(u_d, sq_ud[:, :, None], sq_ud[:, None, :], 10, 256)
    table = jnp.pad(u_d.reshape(bs * n, -1), ((0, 0), (0, 64)))
    nb1 = _sc_gather(table, idx1.reshape(-1))
    x1 = _edge_mlp(u_d, nb1.reshape(bs, n * 10, -1),
                   *(_wt(*dn['c1'][0]) + _wt(*dn['c1'][1])), 10, 256)
    sq_x1 = jnp.sum(x1 * x1, axis=-1)
    w21, b21 = dn['c2'][0]
    c_x1 = x1.shape[2]
    x2 = _edge_gcn_fast(x1, sq_x1[:, :, None], sq_x1[:, None, :],
                        w21[:, :c_x1].T, (w21[:, c_x1:] - w21[:, :c_x1]).T,
                        b21.reshape(1, -1), *_wt(*dn['c2'][1]), 10, 256)

    (w31d, b31d), (w32d, b32d) = dn['c3']
    xcat = jnp.concatenate([x1, x2], axis=2)
    y = _mmg(xcat, w31d.T, b31d.reshape(1, -1))
    yd = _mmg(y, w32d.T, b32d.reshape(1, -1))

    y_pad = jnp.pad(yd, ((0, 0), (2, 2), (0, 0)), mode='edge')
    cat6 = jnp.concatenate([y_pad[:, j:j + n - 1:2, :] for j in range(6)],
                           axis=2)
    w_ds = jnp.transpose(params['ds_conv_w'], (2, 1, 0)).reshape(-1, 128)
    ud_pm = _mmg(cat6, w_ds, params['ds_conv_b'].reshape(1, -1))
    ud_out = jnp.transpose(ud_pm, (0, 2, 1)).reshape(bs, -1, n_stk,
                                                     n_pnt // 2)
    return us_out, ud_out


# speedup vs baseline: 8.7817x; 1.0314x over previous
import functools

import jax
import jax.numpy as jnp
from jax import lax
from jax.experimental import pallas as pl
from jax.experimental.pallas import tpu as pltpu
from jax.experimental.pallas import tpu_sc as plsc

_F32 = jnp.float32


def _wt(w, b):
    return w.T, b.reshape(1, -1)


def _mmg(x, w_t, b):
    bs, n, c = x.shape
    o = w_t.shape[1]

    def body(x_ref, w_ref, b_ref, o_ref):
        o_ref[0] = jax.nn.gelu(
            jnp.dot(x_ref[0], w_ref[...], preferred_element_type=_F32)
            + b_ref[...])

    return pl.pallas_call(
        body,
        grid=(bs,),
        in_specs=[
            pl.BlockSpec((1, n, c), lambda i: (i, 0, 0)),
            pl.BlockSpec((c, o), lambda i: (0, 0)),
            pl.BlockSpec((1, o), lambda i: (0, 0)),
        ],
        out_specs=pl.BlockSpec((1, n, o), lambda i: (i, 0, 0)),
        out_shape=jax.ShapeDtypeStruct((bs, n, o), _F32),
    )(x, w_t, b)


_BF16 = jnp.bfloat16


def _split3(xf):
    hi = xf.astype(_BF16)
    r1 = xf - hi.astype(_F32)
    mid = r1.astype(_BF16)
    lo = (r1 - mid.astype(_F32)).astype(_BF16)
    return hi, mid, lo


def _exact_gather(oh_bf, parts):
    hi, mid, lo = parts
    nb = jnp.dot(oh_bf, hi, preferred_element_type=_F32)
    nb = nb + jnp.dot(oh_bf, mid, preferred_element_type=_F32)
    return nb + jnp.dot(oh_bf, lo, preferred_element_type=_F32)


def _edge_gcn(x, sq_row, sq_col, w1_t, b1, w2_t, b2, k, tile):
    bs, n, c = x.shape
    p = w1_t.shape[1]
    q = w2_t.shape[1]

    def body(xt_ref, xf_ref, sqr_ref, sqc_ref, w1_ref, b1_ref, w2_ref,
             b2_ref, o_ref):
        xt = xt_ref[0]
        xf = xf_ref[0]
        parts = _split3(xf)
        g = lax.dot_general(xt, xf, (((1,), (1,)), ((), ())),
                            preferred_element_type=_F32)
        dist = (sqr_ref[0] - 2.0 * g) + sqc_ref[0]
        iota = lax.broadcasted_iota(jnp.int32, (tile, n), 1)
        acc = jnp.full((tile, q), -jnp.inf, dtype=_F32)
        for _ in range(k):
            idxm = jnp.argmin(dist, axis=1)[:, None]
            oh = iota == idxm
            nb = _exact_gather(oh.astype(_BF16), parts)
            feat = jnp.concatenate([nb - xt, xt], axis=1)
            h = jax.nn.gelu(
                jnp.dot(feat, w1_ref[...], preferred_element_type=_F32)
                + b1_ref[...])
            gl = jax.nn.gelu(jnp.dot(h, w2_ref[...],
                                     preferred_element_type=_F32)
                             + b2_ref[...])
            acc = jnp.maximum(acc, gl)
            dist = jnp.where(oh, jnp.inf, dist)
        o_ref[0] = acc

    return pl.pallas_call(
        body,
        grid=(bs, n // tile),
        in_specs=[
            pl.BlockSpec((1, tile, c), lambda i, j: (i, j, 0)),
            pl.BlockSpec((1, n, c), lambda i, j: (i, 0, 0)),
            pl.BlockSpec((1, tile, 1), lambda i, j: (i, j, 0)),
            pl.BlockSpec((1, 1, n), lambda i, j: (i, 0, 0)),
            pl.BlockSpec((2 * c, p), lambda i, j: (0, 0)),
            pl.BlockSpec((1, p), lambda i, j: (0, 0)),
            pl.BlockSpec((p, q), lambda i, j: (0, 0)),
            pl.BlockSpec((1, q), lambda i, j: (0, 0)),
        ],
        out_specs=pl.BlockSpec((1, tile, q), lambda i, j: (i, j, 0)),
        out_shape=jax.ShapeDtypeStruct((bs, n, q), _F32),
    )(x, x, sq_row, sq_col, w1_t, b1, w2_t, b2)


def _knn_idx(x, sq_row, sq_col, k, tile):
    bs, n, c = x.shape

    def body(xt_ref, xf_ref, sqr_ref, sqc_ref, o_ref):
        g = lax.dot_general(xt_ref[0], xf_ref[0], (((1,), (1,)), ((), ())),
                            preferred_element_type=_F32)
        dist = (sqr_ref[0] - 2.0 * g) + sqc_ref[0]
        iota = lax.broadcasted_iota(jnp.int32, (tile, n), 1)
        base = pl.program_id(0) * n
        cols = []
        for _ in range(k):
            idxm = jnp.argmin(dist, axis=1)[:, None]
            cols.append(idxm + base)
            dist = jnp.where(iota == idxm, jnp.inf, dist)
        o_ref[0] = jnp.concatenate(cols, axis=1)

    return pl.pallas_call(
        body,
        grid=(bs, n // tile),
        in_specs=[
            pl.BlockSpec((1, tile, c), lambda i, j: (i, j, 0)),
            pl.BlockSpec((1, n, c), lambda i, j: (i, 0, 0)),
            pl.BlockSpec((1, tile, 1), lambda i, j: (i, j, 0)),
            pl.BlockSpec((1, 1, n), lambda i, j: (i, 0, 0)),
        ],
        out_specs=pl.BlockSpec((1, tile, k), lambda i, j: (i, j, 0)),
        out_shape=jax.ShapeDtypeStruct((bs, n, k), jnp.int32),
    )(x, x, sq_row, sq_col)


def _sc_gather(table, idx):
    v, d = table.shape
    b = idx.shape[0]
    info = plsc.get_sparse_core_info()
    nw = info.num_cores * info.num_subcores
    b_per_w = b // nw
    chunk = 160
    mesh = plsc.VectorSubcoreMesh(core_axis_name="c", subcore_axis_name="s")

    n_chunk = b_per_w // chunk

    @functools.partial(
        pl.kernel, mesh=mesh,
        out_type=jax.ShapeDtypeStruct((b, d), _F32),
        scratch_types=[
            pltpu.VMEM((chunk,), jnp.int32),
            pltpu.VMEM((chunk, d), _F32),
            pltpu.VMEM((chunk, d), _F32),
            pltpu.SemaphoreType.DMA,
            pltpu.SemaphoreType.DMA,
        ],
    )
    def k(table_hbm, idx_hbm, out_hbm, idx_v, rows_a, rows_b, sem_a, sem_b):
        wid = lax.axis_index("s") * info.num_cores + lax.axis_index("c")
        base = wid * b_per_w
        bufs = [(rows_a, sem_a), (rows_b, sem_b)]
        pltpu.sync_copy(idx_hbm.at[pl.ds(base, chunk)], idx_v)
        cp = pltpu.async_copy(table_hbm.at[idx_v], rows_a, sem_a)
        for ci in range(n_chunk):
            rows, sem = bufs[ci % 2]
            cp.wait()
            if ci + 1 < n_chunk:
                nrows, nsem = bufs[(ci + 1) % 2]
                pltpu.sync_copy(
                    idx_hbm.at[pl.ds(base + (ci + 1) * chunk, chunk)], idx_v)
                cp = pltpu.async_copy(table_hbm.at[idx_v], nrows, nsem)
            pltpu.sync_copy(rows, out_hbm.at[pl.ds(base + ci * chunk, chunk)])

    return k(table, idx)


def _edge_mlp(x, nb, w1_t, b1, w2_t, b2, k, tile):
    bs, n, c = x.shape
    p = w1_t.shape[1]
    q = w2_t.shape[1]

    c_pad = nb.shape[2]

    def body(xt_ref, nb_ref, w1_ref, b1_ref, w2_ref, b2_ref, o_ref):
        xt = xt_ref[0]
        nb3 = nb_ref[0][:, :c].reshape(tile, k, c)
        diff = (nb3 - xt[:, None, :]).reshape(tile * k, c)
        cen = jnp.broadcast_to(xt[:, None, :],
                               (tile, k, c)).reshape(tile * k, c)
        feat = jnp.concatenate([diff, cen], axis=1)
        h = jax.nn.gelu(
            jnp.dot(feat, w1_ref[...], preferred_element_type=_F32)
            + b1_ref[...])
        gl = jax.nn.gelu(jnp.dot(h, w2_ref[...],
                                 preferred_element_type=_F32) + b2_ref[...])
        o_ref[0] = jnp.max(gl.reshape(tile, k, q), axis=1)

    return pl.pallas_call(
        body,
        grid=(bs, n // tile),
        in_specs=[
            pl.BlockSpec((1, tile, c), lambda i, j: (i, j, 0)),
            pl.BlockSpec((1, tile * k, c_pad), lambda i, j: (i, j, 0)),
            pl.BlockSpec((2 * c, p), lambda i, j: (0, 0)),
            pl.BlockSpec((1, p), lambda i, j: (0, 0)),
            pl.BlockSpec((p, q), lambda i, j: (0, 0)),
            pl.BlockSpec((1, q), lambda i, j: (0, 0)),
        ],
        out_specs=pl.BlockSpec((1, tile, q), lambda i, j: (i, j, 0)),
        out_shape=jax.ShapeDtypeStruct((bs, n, q), _F32),
    )(x, nb, w1_t, b1, w2_t, b2)


def _edge_gcn_fast(x, sq_row, sq_col, wa_t, wcm_t, b1, w2_t, b2, k, tile):
    bs, n, c = x.shape
    p = wa_t.shape[1]
    q = w2_t.shape[1]

    def body(xt_ref, xf_ref, sqr_ref, sqc_ref, wa_ref, wc_ref, b1_ref,
             w2_ref, b2_ref, o_ref):
        xt = xt_ref[0]
        xf = xf_ref[0]
        g = lax.dot_general(xt, xf, (((1,), (1,)), ((), ())),
                            preferred_element_type=_F32)
        dist = (sqr_ref[0] - 2.0 * g) + sqc_ref[0]
        a_full = jnp.dot(xf, wa_ref[...],
                         preferred_element_type=_F32).astype(_BF16)
        b_tile = (jnp.dot(xt, wc_ref[...], preferred_element_type=_F32)
                  + b1_ref[...])
        iota = lax.broadcasted_iota(jnp.int32, (tile, n), 1)
        acc = jnp.full((tile, q), -jnp.inf, dtype=_F32)
        for _ in range(k):
            idxm = jnp.argmin(dist, axis=1)[:, None]
            oh = iota == idxm
            nb_a = jnp.dot(oh.astype(_BF16), a_full,
                           preferred_element_type=_F32)
            h = jax.nn.gelu(nb_a + b_tile)
            gl = jax.nn.gelu(jnp.dot(h, w2_ref[...],
                                     preferred_element_type=_F32)
                             + b2_ref[...])
            acc = jnp.maximum(acc, gl)
            dist = jnp.where(oh, jnp.inf, dist)
        o_ref[0] = acc

    return pl.pallas_call(
        body,
        grid=(bs, n // tile),
        in_specs=[
            pl.BlockSpec((1, tile, c), lambda i, j: (i, j, 0)),
            pl.BlockSpec((1, n, c), lambda i, j: (i, 0, 0)),
            pl.BlockSpec((1, tile, 1), lambda i, j: (i, j, 0)),
            pl.BlockSpec((1, 1, n), lambda i, j: (i, 0, 0)),
            pl.BlockSpec((c, p), lambda i, j: (0, 0)),
            pl.BlockSpec((c, p), lambda i, j: (0, 0)),
            pl.BlockSpec((1, p), lambda i, j: (0, 0)),
            pl.BlockSpec((p, q), lambda i, j: (0, 0)),
            pl.BlockSpec((1, q), lambda i, j: (0, 0)),
        ],
        out_specs=pl.BlockSpec((1, tile, q), lambda i, j: (i, j, 0)),
        out_shape=jax.ShapeDtypeStruct((bs, n, q), _F32),
    )(x, x, sq_row, sq_col, wa_t, wcm_t, b1, w2_t, b2)


def _stage1(d5, s, wt_cat, bt, w_d2s_t, b_d2s, n_stk, n_pnt):
    bs, n, c5 = d5.shape
    cd = wt_cat.shape[1]
    cs = s.shape[2]
    u = w_d2s_t.shape[1]

    def body(d5_ref, s_ref, wt_ref, bt_ref, wd2s_ref, bd_ref, us_ref):
        t = jax.nn.gelu(
            jnp.dot(d5_ref[0], wt_ref[...], preferred_element_type=_F32)
            + bt_ref[...])
        spdn = jnp.max(t.reshape(n_stk, n_pnt, cd), axis=1)
        cat = jnp.concatenate([s_ref[0], spdn], axis=1)
        us_ref[0] = jax.nn.gelu(
            jnp.dot(cat, wd2s_ref[...], preferred_element_type=_F32)
            + bd_ref[...])

    return pl.pallas_call(
        body,
        grid=(bs,),
        in_specs=[
            pl.BlockSpec((1, n, c5), lambda i: (i, 0, 0)),
            pl.BlockSpec((1, n_stk, cs), lambda i: (i, 0, 0)),
            pl.BlockSpec((c5, cd), lambda i: (0, 0)),
            pl.BlockSpec((1, cd), lambda i: (0, 0)),
            pl.BlockSpec((cs + cd, u), lambda i: (0, 0)),
            pl.BlockSpec((1, u), lambda i: (0, 0)),
        ],
        out_specs=pl.BlockSpec((1, n_stk, u), lambda i: (i, 0, 0)),
        out_shape=jax.ShapeDtypeStruct((bs, n_stk, u), _F32),
    )(d5, s, wt_cat, bt, w_d2s_t, b_d2s)


def _sparse_gcn(x, sq_row, sq_col, wts, k):
    bs, n, c = x.shape
    (w11, b11, w12, b12, w21, b21, w22, b22, w31, b31, w32, b32) = wts
    q_out = w32.shape[1]

    def gcn_block(xv, sq_t, sq_f, w1_ref, b1_ref, w2_ref, b2_ref):
        parts = _split3(xv)
        if sq_t is None:
            sq_t = jnp.sum(xv * xv, axis=1, keepdims=True)
            ones_c = jnp.ones((1, xv.shape[1]), dtype=_F32)
            y = xv * xv
            yh = y.astype(_BF16)
            r1 = y - yh.astype(_F32)
            ym = r1.astype(_BF16)
            yl = (r1 - ym.astype(_F32)).astype(_BF16)
            ones_b = ones_c.astype(_BF16)
            sq_f = (lax.dot_general(ones_b, yh, (((1,), (1,)), ((), ())),
                                    preferred_element_type=_F32)
                    + lax.dot_general(ones_b, ym, (((1,), (1,)), ((), ())),
                                      preferred_element_type=_F32)
                    + lax.dot_general(ones_b, yl, (((1,), (1,)), ((), ())),
                                      preferred_element_type=_F32))
        g = lax.dot_general(xv, xv, (((1,), (1,)), ((), ())),
                            preferred_element_type=_F32)
        dist = (sq_t - 2.0 * g) + sq_f
        iota = lax.broadcasted_iota(jnp.int32, dist.shape, 1)
        q = w2_ref.shape[1]
        acc = jnp.full((xv.shape[0], q), -jnp.inf, dtype=_F32)
        for _ in range(k):
            idxm = jnp.argmin(dist, axis=1)[:, None]
            oh = iota == idxm
            nb = _exact_gather(oh.astype(_BF16), parts)
            feat = jnp.concatenate([nb - xv, xv], axis=1)
            h = jax.nn.gelu(
                jnp.dot(feat, w1_ref[...], preferred_element_type=_F32)
                + b1_ref[...])
            gl = jax.nn.gelu(jnp.dot(h, w2_ref[...],
                                     preferred_element_type=_F32)
                             + b2_ref[...])
            acc = jnp.maximum(acc, gl)
            dist = jnp.where(oh, jnp.inf, dist)
        return acc

    def body(x_ref, sqr_ref, sqc_ref, w11_r, b11_r, w12_r, b12_r, w21_r,
             b21_r, w22_r, b22_r, w31_r, b31_r, w32_r, b32_r, o_ref):
        xv = x_ref[0]
        x1 = gcn_block(xv, sqr_ref[0], sqc_ref[0], w11_r, b11_r, w12_r, b12_r)
        x2 = gcn_block(x1, None, None, w21_r, b21_r, w22_r, b22_r)
        xc = jnp.concatenate([x1, x2], axis=1)
        y = jax.nn.gelu(
            jnp.dot(xc, w31_r[...], preferred_element_type=_F32)
            + b31_r[...])
        o_ref[0] = jax.nn.gelu(
            jnp.dot(y, w32_r[...], preferred_element_type=_F32) + b32_r[...])

    w_specs = [pl.BlockSpec(w.shape, lambda i: (0, 0)) for w in wts]
    return pl.pallas_call(
        body,
        grid=(bs,),
        in_specs=[pl.BlockSpec((1, n, c), lambda i: (i, 0, 0)),
                  pl.BlockSpec((1, n, 1), lambda i: (i, 0, 0)),
                  pl.BlockSpec((1, 1, n), lambda i: (i, 0, 0))] + w_specs,
        out_specs=pl.BlockSpec((1, n, q_out), lambda i: (i, 0, 0)),
        out_shape=jax.ShapeDtypeStruct((bs, n, q_out), _F32),
    )(x, sq_row, sq_col, *wts)


def kernel(sparse_fea, dense_fea, params):
    bs, c_dn, n_stk, n_pnt = dense_fea.shape
    c_sp = sparse_fea.shape[1]
    n = n_stk * n_pnt

    s_pm = jnp.transpose(sparse_fea, (0, 2, 1))
    d_pm = jnp.transpose(dense_fea.reshape(bs, c_dn, n), (0, 2, 1))

    d_pad = jnp.pad(d_pm, ((0, 0), (2, 2), (0, 0)), mode='edge')
    d5 = jnp.concatenate([d_pad[:, j:j + n, :] for j in range(5)], axis=2)
    wt_cat = jnp.transpose(params['d2s_tconv_w'], (2, 1, 0)).reshape(-1, c_dn)
    bt = params['d2s_tconv_b'].reshape(1, -1)
    w_d2s, b_d2s = params['d2s_mlp'][0]
    w_s2d, b_s2d = params['s2d_mlp'][0]
    u_s = _stage1(d5, s_pm, wt_cat, bt, w_d2s.T, b_d2s.reshape(1, -1),
                  n_stk, n_pnt)
    rep = jnp.broadcast_to(s_pm[:, :, None, :],
                           (bs, n_stk, n_pnt, c_sp)).reshape(bs, n, c_sp)
    cat_d = jnp.concatenate([d_pm, rep], axis=2)
    u_d = _mmg(cat_d, w_s2d.T, b_s2d.reshape(1, -1))

    def edge_prep(layers):
        (w1, b1), (w2, b2) = layers
        return _wt(w1, b1) + _wt(w2, b2)

    sp = params['sp']
    sp_wts = edge_prep(sp['c1']) + edge_prep(sp['c2']) + edge_prep(sp['c3'])
    sq_us = jnp.sum(u_s * u_s, axis=-1)
    us = _sparse_gcn(u_s, sq_us[:, :, None], sq_us[:, None, :], sp_wts, 2)
    us_out = jnp.transpose(us, (0, 2, 1))

    dn = params['dn']
    sq_ud = jnp.sum(u_d * u_d, axis=-1)
    idx1 = _knn_idx(u_d, sq_ud[:, :, None], sq_ud[:, None, :], 10, 512)
    table = jnp.pad(u_d.reshape(bs * n, -1), ((0, 0), (0, 64)))
    nb1 = _sc_gather(table, idx1.reshape(-1))
    x1 = _edge_mlp(u_d, nb1.reshape(bs, n * 10, -1),
                   *(_wt(*dn['c1'][0]) + _wt(*dn['c1'][1])), 10, 512)
    sq_x1 = jnp.sum(x1 * x1, axis=-1)
    w21, b21 = dn['c2'][0]
    c_x1 = x1.shape[2]
    x2 = _edge_gcn_fast(x1, sq_x1[:, :, None], sq_x1[:, None, :],
                        w21[:, :c_x1].T, (w21[:, c_x1:] - w21[:, :c_x1]).T,
                        b21.reshape(1, -1), *_wt(*dn['c2'][1]), 10, 512)

    (w31d, b31d), (w32d, b32d) = dn['c3']
    xcat = jnp.concatenate([x1, x2], axis=2)
    y = _mmg(xcat, w31d.T, b31d.reshape(1, -1))
    yd = _mmg(y, w32d.T, b32d.reshape(1, -1))

    y_pad = jnp.pad(yd, ((0, 0), (2, 2), (0, 0)), mode='edge')
    cat6 = jnp.concatenate([y_pad[:, j:j + n - 1:2, :] for j in range(6)],
                           axis=2)
    w_ds = jnp.transpose(params['ds_conv_w'], (2, 1, 0)).reshape(-1, 128)
    ud_pm = _mmg(cat6, w_ds, params['ds_conv_b'].reshape(1, -1))
    ud_out = jnp.transpose(ud_pm, (0, 2, 1)).reshape(bs, -1, n_stk,
                                                     n_pnt // 2)
    return us_out, ud_out
